# Initial kernel scaffold; baseline (speedup 1.0000x reference)
import jax
import jax.numpy as jnp
from jax.experimental import pallas as pl


def kernel(x, edge_index, batch, Wl1, bl1, Wr1, br1, att1, bias1, Wl2, bl2, Wr2, br2, att2, bias2, Wrel, brel, Wroot, Wlin, blin):
    raise NotImplementedError("write your pallas kernel here")


# speedup vs baseline: 1.0783x; 1.0783x over previous
import functools
import jax
import jax.numpy as jnp
from jax.experimental import pallas as pl

N = 10000
E = 160000
DIN = 128
DH = 64
H1 = 8
NG = 64
RATIO = 0.5


def _matmul_kernel(x_ref, w_ref, b_ref, o_ref):
    o_ref[...] = (
        jnp.dot(x_ref[...], w_ref[...], preferred_element_type=jnp.float32)
        + b_ref[...]
    )


def _proj(x, W, b, blk_rows=1000):
    n, din = x.shape
    dout = W.shape[1]
    return pl.pallas_call(
        _matmul_kernel,
        out_shape=jax.ShapeDtypeStruct((n, dout), jnp.float32),
        grid=(n // blk_rows,),
        in_specs=[
            pl.BlockSpec((blk_rows, din), lambda i: (i, 0)),
            pl.BlockSpec((din, dout), lambda i: (0, 0)),
            pl.BlockSpec((1, dout), lambda i: (0, 0)),
        ],
        out_specs=pl.BlockSpec((blk_rows, dout), lambda i: (i, 0)),
    )(x, W, b.reshape(1, dout))


def _gatv2_nomax(x, s, d, Wl, bl, Wr, br, att, bias, H, C):
    xl = _proj(x, Wl, bl).reshape(N, H, C)
    xr = _proj(x, Wr, br).reshape(N, H, C)
    m = jax.nn.leaky_relu(xl[s] + xr[d], 0.2)
    alpha = (m * att[None, :, :]).sum(-1)
    ex = jnp.exp(alpha)
    den = jax.ops.segment_sum(ex, d, num_segments=N)
    num = jax.ops.segment_sum(xl[s] * ex[:, :, None], d, num_segments=N)
    out = num / (den[:, :, None] + 1e-16)
    return out.reshape(N, H * C) + bias


def _sortable_i32(f):
    i = jax.lax.bitcast_convert_type(f, jnp.int32)
    return jnp.where(i >= 0, i, i ^ jnp.int32(0x7FFFFFFF))


def _topk_mask(score, batch, counts):
    k = jnp.ceil(RATIO * counts.astype(jnp.float32)).astype(jnp.int32)
    key = _sortable_i32(score)
    kb = key
    lo = jnp.full((NG,), jnp.iinfo(jnp.int32).min, jnp.int32)
    hi = jnp.full((NG,), jnp.iinfo(jnp.int32).max, jnp.int32)

    def body(_, lohi):
        lo, hi = lohi
        mid = lo + ((hi - lo) >> 1)
        cnt = jax.ops.segment_sum(
            (kb > mid[batch]).astype(jnp.int32), batch, num_segments=NG
        )
        smaller = cnt < k
        hi = jnp.where(smaller, mid, hi)
        lo = jnp.where(smaller, lo, mid + 1)
        return lo, hi

    lo, hi = jax.lax.fori_loop(0, 32, body, (lo, hi))
    v = lo
    n_greater = jax.ops.segment_sum(
        (kb > v[batch]).astype(jnp.int32), batch, num_segments=NG
    )
    need = k - n_greater
    tie = (kb == v[batch]).astype(jnp.int32)
    cs = jnp.cumsum(tie)
    tie_per_g = jax.ops.segment_sum(tie, batch, num_segments=NG)
    excl = jnp.cumsum(tie_per_g) - tie_per_g
    tie_rank = cs - excl[batch]
    kept = (kb > v[batch]) | ((tie == 1) & (tie_rank <= need[batch]))
    return kept, k


def kernel(x, edge_index, batch, Wl1, bl1, Wr1, br1, att1, bias1,
           Wl2, bl2, Wr2, br2, att2, bias2, Wrel, brel, Wroot, Wlin, blin):
    src, dst = edge_index[0], edge_index[1]
    loop = jnp.arange(N, dtype=src.dtype)
    s = jnp.concatenate([src, loop])
    d = jnp.concatenate([dst, loop])

    h = jax.nn.elu(_gatv2_nomax(x, s, d, Wl1, bl1, Wr1, br1, att1, bias1, H1, DH))
    h = jax.nn.selu(_gatv2_nomax(h, s, d, Wl2, bl2, Wr2, br2, att2, bias2, 1, DH))

    agg = jax.ops.segment_sum(h[src], dst, num_segments=N)
    score = jnp.tanh((agg @ Wrel + brel + h @ Wroot).reshape(-1))

    counts = jax.ops.segment_sum(jnp.ones((N,), jnp.int32), batch, num_segments=NG)
    kept, _ = _topk_mask(score, batch, counts)

    xo = h * score[:, None]
    xp = jnp.where(kept[:, None], xo, 0.0)
    xm = jnp.where(kept[:, None], xo, -jnp.inf)
    maxp = jax.ops.segment_max(xm, batch, num_segments=NG)
    maxp = jnp.where(jnp.isfinite(maxp), maxp, 0.0)
    sump = jax.ops.segment_sum(xp, batch, num_segments=NG)
    cnt = jax.ops.segment_sum(kept.astype(jnp.float32), batch, num_segments=NG)
    meanp = sump / jnp.clip(cnt, 1.0, None)[:, None]
    feat = jnp.concatenate([maxp, meanp], axis=1)
    out = jax.nn.relu(feat @ Wlin + blin)
    return jax.nn.log_softmax(out, axis=-1)


# speedup vs baseline: 1.1307x; 1.0485x over previous
import functools
import jax
import jax.numpy as jnp
from jax import lax
from jax.experimental import pallas as pl
from jax.experimental.pallas import tpu as pltpu
from jax.experimental.pallas import tpu_sc as plsc

N = 10000
E = 160000
DIN = 128
DH = 64
H1 = 8
NG = 64
RATIO = 0.5

EALL = E + N
NW = 32
BE = 32
NBATCH = 168
EPW = BE * NBATCH
EP = NW * EPW

_D1 = H1 * DH


def _matmul_kernel(x_ref, w_ref, b_ref, o_ref):
    o_ref[...] = (
        jnp.dot(x_ref[...], w_ref[...], preferred_element_type=jnp.float32)
        + b_ref[...]
    )


def _proj(x, W, b, blk_rows=1000):
    n, din = x.shape
    dout = W.shape[1]
    return pl.pallas_call(
        _matmul_kernel,
        out_shape=jax.ShapeDtypeStruct((n, dout), jnp.float32),
        grid=(n // blk_rows,),
        in_specs=[
            pl.BlockSpec((blk_rows, din), lambda i: (i, 0)),
            pl.BlockSpec((din, dout), lambda i: (0, 0)),
            pl.BlockSpec((1, dout), lambda i: (0, 0)),
        ],
        out_specs=pl.BlockSpec((blk_rows, dout), lambda i: (i, 0)),
    )(x, W, b.reshape(1, dout))


def _sc_attn1_body(xl_hbm, xr_hbm, s2_hbm, d2_hbm, att_hbm, w_hbm,
                   sidx_v, didx_v, att_v, glb, grb, wbuf,
                   gl_sem, gr_sem, w_sem):
    wid = lax.axis_index("s") * 2 + lax.axis_index("c")
    base = wid * EPW
    rowb = wid * NBATCH

    pltpu.sync_copy(s2_hbm.at[pl.ds(rowb, NBATCH)], sidx_v)
    pltpu.sync_copy(d2_hbm.at[pl.ds(rowb, NBATCH)], didx_v)
    pltpu.sync_copy(att_hbm, att_v)

    lane = lax.iota(jnp.int32, 16)
    att_regs = [att_v[pl.ds(k * 16, 16)] for k in range(_D1 // 16)]

    def gdesc(j, slot):
        return (
            pltpu.make_async_copy(xl_hbm.at[sidx_v.at[j]], glb.at[slot],
                                  gl_sem.at[slot]),
            pltpu.make_async_copy(xr_hbm.at[didx_v.at[j]], grb.at[slot],
                                  gr_sem.at[slot]),
        )

    def wdesc(j, slot):
        return pltpu.make_async_copy(
            wbuf.at[slot], w_hbm.at[pl.ds((base + j * BE) * H1, BE * H1)],
            w_sem.at[slot])

    for b in range(2):
        for dsc in gdesc(b, b):
            dsc.start()

    def compute_batch(j, slot):
        for dsc in gdesc(j, slot):
            dsc.wait()

        @pl.when(j >= 2)
        def _():
            wdesc(j - 2, slot).wait()

        perms = [lane ^ sh for sh in (8, 4, 2, 1)]

        def hsum(v):
            for pm in perms:
                v = v + v.at[pm].get(mode="promise_in_bounds")
            return v

        def pair_body(p, _):
            e0 = 2 * p
            alpha = jnp.zeros((16,), jnp.float32)
            for ei in range(2):
                for h in range(H1):
                    acc = jnp.zeros((16,), jnp.float32)
                    for c4 in range(4):
                        k = h * 4 + c4
                        sl = pl.ds(k * 16, 16)
                        z = glb[slot, e0 + ei, sl] + grb[slot, e0 + ei, sl]
                        m = jnp.maximum(z, 0.2 * z)
                        acc = acc + m * att_regs[k]
                    alpha = jnp.where(lane == ei * 8 + h, hsum(acc), alpha)
            w16 = jnp.exp(alpha)
            eid = base + j * BE + e0 + (lane >> 3)
            w16 = jnp.where(eid < EALL, w16, 0.0)
            wbuf[slot, pl.ds(p * 16, 16)] = w16
            return 0

        lax.fori_loop(0, BE // 2, pair_body, 0)
        wdesc(j, slot).start()

        @pl.when(j + 2 < NBATCH)
        def _():
            for dsc in gdesc(j + 2, slot):
                dsc.start()

    def outer(i, _):
        for b in range(2):
            compute_batch(2 * i + b, b)
        return 0

    lax.fori_loop(0, NBATCH // 2, outer, 0)
    for b in range(2):
        wdesc(NBATCH - 2 + b, b).wait()


@functools.partial(jax.jit, static_argnums=())
def _sc_attn1(xl, xr, s2, d2, att_flat):
    mesh = plsc.VectorSubcoreMesh(core_axis_name="c", subcore_axis_name="s")
    f = pl.kernel(
        _sc_attn1_body,
        out_type=jax.ShapeDtypeStruct((EP * H1,), jnp.float32),
        mesh=mesh,
        scratch_types=[
            pltpu.VMEM((NBATCH, BE), jnp.int32),
            pltpu.VMEM((NBATCH, BE), jnp.int32),
            pltpu.VMEM((_D1,), jnp.float32),
            pltpu.VMEM((2, BE, _D1), jnp.float32),
            pltpu.VMEM((2, BE, _D1), jnp.float32),
            pltpu.VMEM((2, BE * H1), jnp.float32),
            pltpu.SemaphoreType.DMA((2,)),
            pltpu.SemaphoreType.DMA((2,)),
            pltpu.SemaphoreType.DMA((2,)),
        ],
    )
    return f(xl, xr, s2, d2, att_flat)


def _sortable_i32(f):
    i = lax.bitcast_convert_type(f, jnp.int32)
    return jnp.where(i >= 0, i, i ^ jnp.int32(0x7FFFFFFF))


def _topk_mask(score, batch, counts):
    k = jnp.ceil(RATIO * counts.astype(jnp.float32)).astype(jnp.int32)
    kb = _sortable_i32(score)
    lo = jnp.full((NG,), jnp.iinfo(jnp.int32).min, jnp.int32)
    hi = jnp.full((NG,), jnp.iinfo(jnp.int32).max, jnp.int32)

    def body(_, lohi):
        lo, hi = lohi
        mid = lo + ((hi - lo) >> 1)
        cnt = jax.ops.segment_sum(
            (kb > mid[batch]).astype(jnp.int32), batch, num_segments=NG)
        smaller = cnt < k
        hi = jnp.where(smaller, mid, hi)
        lo = jnp.where(smaller, lo, mid + 1)
        return lo, hi

    lo, hi = lax.fori_loop(0, 32, body, (lo, hi))
    v = lo
    n_greater = jax.ops.segment_sum(
        (kb > v[batch]).astype(jnp.int32), batch, num_segments=NG)
    need = k - n_greater
    tie = (kb == v[batch]).astype(jnp.int32)
    cs = jnp.cumsum(tie)
    tie_per_g = jax.ops.segment_sum(tie, batch, num_segments=NG)
    excl = jnp.cumsum(tie_per_g) - tie_per_g
    tie_rank = cs - excl[batch]
    kept = (kb > v[batch]) | ((tie == 1) & (tie_rank <= need[batch]))
    return kept


def kernel(x, edge_index, batch, Wl1, bl1, Wr1, br1, att1, bias1,
           Wl2, bl2, Wr2, br2, att2, bias2, Wrel, brel, Wroot, Wlin, blin):
    src, dst = edge_index[0], edge_index[1]
    loop = jnp.arange(N, dtype=src.dtype)
    s = jnp.concatenate([src, loop])
    d = jnp.concatenate([dst, loop])
    pad = jnp.zeros((EP - EALL,), jnp.int32)
    s2 = jnp.concatenate([s, pad]).reshape(EP // BE, BE)
    d2 = jnp.concatenate([d, pad]).reshape(EP // BE, BE)

    xl = _proj(x, Wl1, bl1)
    xr = _proj(x, Wr1, br1)
    w1 = _sc_attn1(xl, xr, s2, d2, att1.reshape(_D1))
    ex = w1.reshape(EP, H1)[:EALL]

    xl3 = xl.reshape(N, H1, DH)
    den = jax.ops.segment_sum(ex, d, num_segments=N)
    num = jax.ops.segment_sum(xl3[s] * ex[:, :, None], d, num_segments=N)
    out = num / (den[:, :, None] + 1e-16)
    h = jax.nn.elu(out.reshape(N, _D1) + bias1)

    xl2 = _proj(h, Wl2, bl2, blk_rows=1000)
    xr2 = _proj(h, Wr2, br2, blk_rows=1000)
    m2 = jax.nn.leaky_relu(xl2[s] + xr2[d], 0.2)
    alpha2 = (m2 * att2[None, 0, :]).sum(-1)
    ex2 = jnp.exp(alpha2)
    den2 = jax.ops.segment_sum(ex2, d, num_segments=N)
    num2 = jax.ops.segment_sum(xl2[s] * ex2[:, None], d, num_segments=N)
    h = jax.nn.selu(num2 / (den2[:, None] + 1e-16) + bias2)

    agg = jax.ops.segment_sum(h[src], dst, num_segments=N)
    score = jnp.tanh((agg @ Wrel + brel + h @ Wroot).reshape(-1))

    counts = jax.ops.segment_sum(jnp.ones((N,), jnp.int32), batch, num_segments=NG)
    kept = _topk_mask(score, batch, counts)

    xo = h * score[:, None]
    xp = jnp.where(kept[:, None], xo, 0.0)
    xm = jnp.where(kept[:, None], xo, -jnp.inf)
    maxp = jax.ops.segment_max(xm, batch, num_segments=NG)
    maxp = jnp.where(jnp.isfinite(maxp), maxp, 0.0)
    sump = jax.ops.segment_sum(xp, batch, num_segments=NG)
    cnt = jax.ops.segment_sum(kept.astype(jnp.float32), batch, num_segments=NG)
    meanp = sump / jnp.clip(cnt, 1.0, None)[:, None]
    feat = jnp.concatenate([maxp, meanp], axis=1)
    out = jax.nn.relu(feat @ Wlin + blin)
    return jax.nn.log_softmax(out, axis=-1)


# speedup vs baseline: 4.2552x; 3.7634x over previous
import functools
import jax
import jax.numpy as jnp
from jax import lax
from jax.experimental import pallas as pl
from jax.experimental.pallas import tpu as pltpu
from jax.experimental.pallas import tpu_sc as plsc

N = 10000
E = 160000
DIN = 128
DH = 64
H1 = 8
NG = 64
RATIO = 0.5

EALL = E + N
NW = 32
BE = 32
NBATCH = 168
EPW = BE * NBATCH
EP = NW * EPW

_D1 = H1 * DH


def _matmul_kernel(x_ref, w_ref, b_ref, o_ref):
    o_ref[...] = (
        jnp.dot(x_ref[...], w_ref[...], preferred_element_type=jnp.float32)
        + b_ref[...]
    )


def _proj(x, W, b, blk_rows=1000):
    n, din = x.shape
    dout = W.shape[1]
    return pl.pallas_call(
        _matmul_kernel,
        out_shape=jax.ShapeDtypeStruct((n, dout), jnp.float32),
        grid=(n // blk_rows,),
        in_specs=[
            pl.BlockSpec((blk_rows, din), lambda i: (i, 0)),
            pl.BlockSpec((din, dout), lambda i: (0, 0)),
            pl.BlockSpec((1, dout), lambda i: (0, 0)),
        ],
        out_specs=pl.BlockSpec((blk_rows, dout), lambda i: (i, 0)),
    )(x, W, b.reshape(1, dout))


def _sc_attn1_body(xl_hbm, xr_hbm, s2_hbm, d2_hbm, att_hbm, w_hbm,
                   sidx_v, didx_v, att_v, glb, grb, wbuf,
                   gl_sem, gr_sem, w_sem):
    wid = lax.axis_index("s") * 2 + lax.axis_index("c")
    base = wid * EPW
    rowb = wid * NBATCH

    pltpu.sync_copy(s2_hbm.at[pl.ds(rowb, NBATCH)], sidx_v)
    pltpu.sync_copy(d2_hbm.at[pl.ds(rowb, NBATCH)], didx_v)
    pltpu.sync_copy(att_hbm, att_v)

    lane = lax.iota(jnp.int32, 16)
    att_regs = [att_v[pl.ds(k * 16, 16)] for k in range(_D1 // 16)]

    def gdesc(j, slot):
        return (
            pltpu.make_async_copy(xl_hbm.at[sidx_v.at[j]], glb.at[slot],
                                  gl_sem.at[slot]),
            pltpu.make_async_copy(xr_hbm.at[didx_v.at[j]], grb.at[slot],
                                  gr_sem.at[slot]),
        )

    def wdesc(j, slot):
        return pltpu.make_async_copy(
            wbuf.at[slot], w_hbm.at[pl.ds((base + j * BE) * H1, BE * H1)],
            w_sem.at[slot])

    for b in range(2):
        for dsc in gdesc(b, b):
            dsc.start()

    def compute_batch(j, slot):
        for dsc in gdesc(j, slot):
            dsc.wait()

        @pl.when(j >= 2)
        def _():
            wdesc(j - 2, slot).wait()

        perms = [lane ^ sh for sh in (8, 4, 2, 1)]

        def hsum(v):
            for pm in perms:
                v = v + v.at[pm].get(mode="promise_in_bounds")
            return v

        def pair_body(p, _):
            e0 = 2 * p
            alpha = jnp.zeros((16,), jnp.float32)
            for ei in range(2):
                for h in range(H1):
                    acc = jnp.zeros((16,), jnp.float32)
                    for c4 in range(4):
                        k = h * 4 + c4
                        sl = pl.ds(k * 16, 16)
                        z = glb[slot, e0 + ei, sl] + grb[slot, e0 + ei, sl]
                        m = jnp.maximum(z, 0.2 * z)
                        acc = acc + m * att_regs[k]
                    alpha = jnp.where(lane == ei * 8 + h, hsum(acc), alpha)
            w16 = jnp.exp(alpha)
            eid = base + j * BE + e0 + (lane >> 3)
            w16 = jnp.where(eid < EALL, w16, 0.0)
            wbuf[slot, pl.ds(p * 16, 16)] = w16
            return 0

        lax.fori_loop(0, BE // 2, pair_body, 0)
        wdesc(j, slot).start()

        @pl.when(j + 2 < NBATCH)
        def _():
            for dsc in gdesc(j + 2, slot):
                dsc.start()

    def outer(i, _):
        for b in range(2):
            compute_batch(2 * i + b, b)
        return 0

    lax.fori_loop(0, NBATCH // 2, outer, 0)
    for b in range(2):
        wdesc(NBATCH - 2 + b, b).wait()


@functools.partial(jax.jit, static_argnums=())
def _sc_attn1(xl, xr, s2, d2, att_flat):
    mesh = plsc.VectorSubcoreMesh(core_axis_name="c", subcore_axis_name="s")
    f = pl.kernel(
        _sc_attn1_body,
        out_type=jax.ShapeDtypeStruct((EP * H1,), jnp.float32),
        mesh=mesh,
        scratch_types=[
            pltpu.VMEM((NBATCH, BE), jnp.int32),
            pltpu.VMEM((NBATCH, BE), jnp.int32),
            pltpu.VMEM((_D1,), jnp.float32),
            pltpu.VMEM((2, BE, _D1), jnp.float32),
            pltpu.VMEM((2, BE, _D1), jnp.float32),
            pltpu.VMEM((2, BE * H1), jnp.float32),
            pltpu.SemaphoreType.DMA((2,)),
            pltpu.SemaphoreType.DMA((2,)),
            pltpu.SemaphoreType.DMA((2,)),
        ],
    )
    return f(xl, xr, s2, d2, att_flat)


_RW = 128
_NPAD = 10240
_RPT = _NPAD // 16


def _sc_scat1_body(xl4_hbm, w_hbm, s2_hbm, d2_hbm, out4_hbm,
                   sidx_v, didx_v, ibuf, gbuf, wv, cbuf, zbuf, acc,
                   g_sem, w_sem, s_sem):
    cid = lax.axis_index("c")
    tid = lax.axis_index("s")
    wid = tid * 2 + cid
    base = wid * EPW
    rowb = wid * NBATCH
    lane = lax.iota(jnp.int32, 16)

    pltpu.sync_copy(s2_hbm.at[pl.ds(rowb, NBATCH)], sidx_v)
    pltpu.sync_copy(d2_hbm.at[pl.ds(rowb, NBATCH)], didx_v)

    def zrow(r, _):
        for c9 in range(_RW // 16):
            zbuf[r, pl.ds(c9 * 16, 16)] = jnp.zeros((16,), jnp.float32)
        return 0

    lax.fori_loop(0, 128, zrow, 0)

    for p in range(4):
        for z5 in range(5):
            pltpu.sync_copy(zbuf, acc.at[pl.ds(tid * _RPT + z5 * 128, 128)])
        plsc.subcore_barrier()

        def idx_compute(j, slot):
            a = sidx_v[j, pl.ds(0, 16)] * 4 + p
            b = sidx_v[j, pl.ds(16, 16)] * 4 + p
            ibuf[slot, pl.ds(0, 16)] = a
            ibuf[slot, pl.ds(16, 16)] = b

        def gdesc(j, slot):
            return (
                pltpu.make_async_copy(xl4_hbm.at[ibuf.at[slot]],
                                      gbuf.at[slot], g_sem.at[slot]),
                pltpu.make_async_copy(
                    w_hbm.at[pl.ds((base + j * BE) * H1, BE * H1)],
                    wv.at[slot], w_sem.at[slot]),
            )

        def sstart(j, slot):
            pltpu.async_copy(cbuf.at[slot], acc.at[didx_v.at[j]],
                             s_sem.at[slot], add=True)

        def swait(j, slot):
            pltpu.make_async_copy(cbuf.at[slot], acc.at[didx_v.at[j]],
                                  s_sem.at[slot]).wait()

        for b in range(2):
            idx_compute(b, b)
            for dsc in gdesc(b, b):
                dsc.start()

        def compute_batch(j, slot):
            @pl.when(j >= 2)
            def _():
                swait(j - 2, slot)

            for dsc in gdesc(j, slot):
                dsc.wait()

            def ebody(ep, _):
                wpair = wv[slot, pl.ds(ep * 16, 16)]
                for ei in range(2):
                    e = 2 * ep + ei
                    w0 = wpair[ei * H1 + 2 * p]
                    w1 = wpair[ei * H1 + 2 * p + 1]
                    for c8 in range(8):
                        sl = pl.ds(c8 * 16, 16)
                        ws = w0 if c8 < 4 else w1
                        cbuf[slot, e, sl] = gbuf[slot, e, sl] * ws
                return 0

            lax.fori_loop(0, BE // 2, ebody, 0)
            sstart(j, slot)

            @pl.when(j + 2 < NBATCH)
            def _():
                idx_compute(j + 2, slot)
                for dsc in gdesc(j + 2, slot):
                    dsc.start()

        def outer(i, _):
            for b in range(2):
                compute_batch(2 * i + b, b)
            return 0

        lax.fori_loop(0, NBATCH // 2, outer, 0)
        for b in range(2):
            swait(NBATCH - 2 + b, b)
        plsc.subcore_barrier()
        pltpu.sync_copy(acc.at[pl.ds(tid * _RPT, _RPT)],
                        out4_hbm.at[p, cid, pl.ds(tid * _RPT, _RPT)])


def _sc_scat1(xl4, w, s2, d2):
    mesh = plsc.VectorSubcoreMesh(core_axis_name="c", subcore_axis_name="s")
    f = pl.kernel(
        _sc_scat1_body,
        out_type=jax.ShapeDtypeStruct((4, 2, _NPAD, _RW), jnp.float32),
        mesh=mesh,
        compiler_params=pltpu.CompilerParams(use_tc_tiling_on_sc=False),
        scratch_types=[
            pltpu.VMEM((NBATCH, BE), jnp.int32),
            pltpu.VMEM((NBATCH, BE), jnp.int32),
            pltpu.VMEM((2, BE), jnp.int32),
            pltpu.VMEM((2, BE, 128), jnp.float32),
            pltpu.VMEM((2, BE * H1), jnp.float32),
            pltpu.VMEM((2, BE, _RW), jnp.float32),
            pltpu.VMEM((128, _RW), jnp.float32),
            pltpu.VMEM_SHARED((_NPAD, _RW), jnp.float32),
            pltpu.SemaphoreType.DMA((2,)),
            pltpu.SemaphoreType.DMA((2,)),
            pltpu.SemaphoreType.DMA((2,)),
        ],
    )
    return f(xl4, w, s2, d2)


def _sortable_i32(f):
    i = lax.bitcast_convert_type(f, jnp.int32)
    return jnp.where(i >= 0, i, i ^ jnp.int32(0x7FFFFFFF))


def _topk_mask(score, batch, counts):
    k = jnp.ceil(RATIO * counts.astype(jnp.float32)).astype(jnp.int32)
    kb = _sortable_i32(score)
    lo = jnp.full((NG,), jnp.iinfo(jnp.int32).min, jnp.int32)
    hi = jnp.full((NG,), jnp.iinfo(jnp.int32).max, jnp.int32)

    def body(_, lohi):
        lo, hi = lohi
        mid = lo + ((hi - lo) >> 1)
        cnt = jax.ops.segment_sum(
            (kb > mid[batch]).astype(jnp.int32), batch, num_segments=NG)
        smaller = cnt < k
        hi = jnp.where(smaller, mid, hi)
        lo = jnp.where(smaller, lo, mid + 1)
        return lo, hi

    lo, hi = lax.fori_loop(0, 32, body, (lo, hi))
    v = lo
    n_greater = jax.ops.segment_sum(
        (kb > v[batch]).astype(jnp.int32), batch, num_segments=NG)
    need = k - n_greater
    tie = (kb == v[batch]).astype(jnp.int32)
    cs = jnp.cumsum(tie)
    tie_per_g = jax.ops.segment_sum(tie, batch, num_segments=NG)
    excl = jnp.cumsum(tie_per_g) - tie_per_g
    tie_rank = cs - excl[batch]
    kept = (kb > v[batch]) | ((tie == 1) & (tie_rank <= need[batch]))
    return kept


def kernel(x, edge_index, batch, Wl1, bl1, Wr1, br1, att1, bias1,
           Wl2, bl2, Wr2, br2, att2, bias2, Wrel, brel, Wroot, Wlin, blin):
    src, dst = edge_index[0], edge_index[1]
    loop = jnp.arange(N, dtype=src.dtype)
    s = jnp.concatenate([src, loop])
    d = jnp.concatenate([dst, loop])
    pad = jnp.zeros((EP - EALL,), jnp.int32)
    s2 = jnp.concatenate([s, pad]).reshape(EP // BE, BE)
    d2 = jnp.concatenate([d, pad]).reshape(EP // BE, BE)

    xl = _proj(x, Wl1, bl1)
    xr = _proj(x, Wr1, br1)
    w1 = _sc_attn1(xl, xr, s2, d2, att1.reshape(_D1))
    out4 = _sc_scat1(xl.reshape(N * 4, 128), w1, s2, d2)
    comb = (out4[:, 0] + out4[:, 1])[:, :N]
    num = comb.transpose(1, 0, 2).reshape(N, H1, DH)
    ex = w1.reshape(EP, H1)[:EALL]
    den = jax.ops.segment_sum(ex, d, num_segments=N)
    out = num / (den[:, :, None] + 1e-16)
    h = jax.nn.elu(out.reshape(N, _D1) + bias1)

    xl2 = _proj(h, Wl2, bl2, blk_rows=1000)
    xr2 = _proj(h, Wr2, br2, blk_rows=1000)
    m2 = jax.nn.leaky_relu(xl2[s] + xr2[d], 0.2)
    alpha2 = (m2 * att2[None, 0, :]).sum(-1)
    ex2 = jnp.exp(alpha2)
    den2 = jax.ops.segment_sum(ex2, d, num_segments=N)
    num2 = jax.ops.segment_sum(xl2[s] * ex2[:, None], d, num_segments=N)
    h = jax.nn.selu(num2 / (den2[:, None] + 1e-16) + bias2)

    agg = jax.ops.segment_sum(h[src], dst, num_segments=N)
    score = jnp.tanh((agg @ Wrel + brel + h @ Wroot).reshape(-1))

    counts = jax.ops.segment_sum(jnp.ones((N,), jnp.int32), batch, num_segments=NG)
    kept = _topk_mask(score, batch, counts)

    xo = h * score[:, None]
    xp = jnp.where(kept[:, None], xo, 0.0)
    xm = jnp.where(kept[:, None], xo, -jnp.inf)
    maxp = jax.ops.segment_max(xm, batch, num_segments=NG)
    maxp = jnp.where(jnp.isfinite(maxp), maxp, 0.0)
    sump = jax.ops.segment_sum(xp, batch, num_segments=NG)
    cnt = jax.ops.segment_sum(kept.astype(jnp.float32), batch, num_segments=NG)
    meanp = sump / jnp.clip(cnt, 1.0, None)[:, None]
    feat = jnp.concatenate([maxp, meanp], axis=1)
    out = jax.nn.relu(feat @ Wlin + blin)
    return jax.nn.log_softmax(out, axis=-1)


# speedup vs baseline: 6.1299x; 1.4406x over previous
import functools
import jax
import jax.numpy as jnp
from jax import lax
from jax.experimental import pallas as pl
from jax.experimental.pallas import tpu as pltpu
from jax.experimental.pallas import tpu_sc as plsc

N = 10000
E = 160000
DIN = 128
DH = 64
H1 = 8
NG = 64
RATIO = 0.5

EALL = E + N
NW = 32
BE = 32
NBATCH = 168
EPW = BE * NBATCH
EP = NW * EPW

_D1 = H1 * DH


def _matmul_kernel(x_ref, w_ref, b_ref, o_ref):
    o_ref[...] = (
        jnp.dot(x_ref[...], w_ref[...], preferred_element_type=jnp.float32)
        + b_ref[...]
    )


def _proj(x, W, b, blk_rows=1000):
    n, din = x.shape
    dout = W.shape[1]
    return pl.pallas_call(
        _matmul_kernel,
        out_shape=jax.ShapeDtypeStruct((n, dout), jnp.float32),
        grid=(n // blk_rows,),
        in_specs=[
            pl.BlockSpec((blk_rows, din), lambda i: (i, 0)),
            pl.BlockSpec((din, dout), lambda i: (0, 0)),
            pl.BlockSpec((1, dout), lambda i: (0, 0)),
        ],
        out_specs=pl.BlockSpec((blk_rows, dout), lambda i: (i, 0)),
    )(x, W, b.reshape(1, dout))


def _sc_attn1_body(xl_hbm, xr_hbm, s2_hbm, d2_hbm, att_hbm, w_hbm,
                   sidx_v, didx_v, att_v, glb, grb, wbuf,
                   gl_sem, gr_sem, w_sem):
    wid = lax.axis_index("s") * 2 + lax.axis_index("c")
    base = wid * EPW
    rowb = wid * NBATCH

    pltpu.sync_copy(s2_hbm.at[pl.ds(rowb, NBATCH)], sidx_v)
    pltpu.sync_copy(d2_hbm.at[pl.ds(rowb, NBATCH)], didx_v)
    pltpu.sync_copy(att_hbm, att_v)

    lane = lax.iota(jnp.int32, 16)
    att_regs = [att_v[pl.ds(k * 16, 16)] for k in range(_D1 // 16)]

    def gdesc(j, slot):
        return (
            pltpu.make_async_copy(xl_hbm.at[sidx_v.at[j]], glb.at[slot],
                                  gl_sem.at[slot]),
            pltpu.make_async_copy(xr_hbm.at[didx_v.at[j]], grb.at[slot],
                                  gr_sem.at[slot]),
        )

    def wdesc(j, slot):
        return pltpu.make_async_copy(
            wbuf.at[slot], w_hbm.at[pl.ds((base + j * BE) * H1, BE * H1)],
            w_sem.at[slot])

    for b in range(2):
        for dsc in gdesc(b, b):
            dsc.start()

    def compute_batch(j, slot):
        for dsc in gdesc(j, slot):
            dsc.wait()

        @pl.when(j >= 2)
        def _():
            wdesc(j - 2, slot).wait()

        perms = [lane ^ sh for sh in (8, 4, 2, 1)]

        def hsum(v):
            for pm in perms:
                v = v + v.at[pm].get(mode="promise_in_bounds")
            return v

        def pair_body(p, _):
            e0 = 2 * p
            alpha = jnp.zeros((16,), jnp.float32)
            for ei in range(2):
                for h in range(H1):
                    acc = jnp.zeros((16,), jnp.float32)
                    for c4 in range(4):
                        k = h * 4 + c4
                        sl = pl.ds(k * 16, 16)
                        z = glb[slot, e0 + ei, sl] + grb[slot, e0 + ei, sl]
                        m = jnp.maximum(z, 0.2 * z)
                        acc = acc + m * att_regs[k]
                    alpha = jnp.where(lane == ei * 8 + h, hsum(acc), alpha)
            w16 = jnp.exp(alpha)
            eid = base + j * BE + e0 + (lane >> 3)
            w16 = jnp.where(eid < EALL, w16, 0.0)
            wbuf[slot, pl.ds(p * 16, 16)] = w16
            return 0

        lax.fori_loop(0, BE // 2, pair_body, 0)
        wdesc(j, slot).start()

        @pl.when(j + 2 < NBATCH)
        def _():
            for dsc in gdesc(j + 2, slot):
                dsc.start()

    def outer(i, _):
        for b in range(2):
            compute_batch(2 * i + b, b)
        return 0

    lax.fori_loop(0, NBATCH // 2, outer, 0)
    for b in range(2):
        wdesc(NBATCH - 2 + b, b).wait()


@functools.partial(jax.jit, static_argnums=())
def _sc_attn1(xl, xr, s2, d2, att_flat):
    mesh = plsc.VectorSubcoreMesh(core_axis_name="c", subcore_axis_name="s")
    f = pl.kernel(
        _sc_attn1_body,
        out_type=jax.ShapeDtypeStruct((EP * H1,), jnp.float32),
        mesh=mesh,
        scratch_types=[
            pltpu.VMEM((NBATCH, BE), jnp.int32),
            pltpu.VMEM((NBATCH, BE), jnp.int32),
            pltpu.VMEM((_D1,), jnp.float32),
            pltpu.VMEM((2, BE, _D1), jnp.float32),
            pltpu.VMEM((2, BE, _D1), jnp.float32),
            pltpu.VMEM((2, BE * H1), jnp.float32),
            pltpu.SemaphoreType.DMA((2,)),
            pltpu.SemaphoreType.DMA((2,)),
            pltpu.SemaphoreType.DMA((2,)),
        ],
    )
    return f(xl, xr, s2, d2, att_flat)


_RW = 128
_NPAD = 10240
_RPT = _NPAD // 16


def _sc_scat1_body(xl4_hbm, w_hbm, s2_hbm, d2_hbm, out4_hbm,
                   sidx_v, didx_v, ibuf, gbuf, wv, cbuf, zbuf, acc,
                   g_sem, w_sem, s_sem):
    cid = lax.axis_index("c")
    tid = lax.axis_index("s")
    wid = tid * 2 + cid
    base = wid * EPW
    rowb = wid * NBATCH
    lane = lax.iota(jnp.int32, 16)

    pltpu.sync_copy(s2_hbm.at[pl.ds(rowb, NBATCH)], sidx_v)
    pltpu.sync_copy(d2_hbm.at[pl.ds(rowb, NBATCH)], didx_v)

    def zrow(r, _):
        for c9 in range(_RW // 16):
            zbuf[r, pl.ds(c9 * 16, 16)] = jnp.zeros((16,), jnp.float32)
        return 0

    lax.fori_loop(0, 128, zrow, 0)

    for p in range(4):
        for z5 in range(5):
            pltpu.sync_copy(zbuf, acc.at[pl.ds(tid * _RPT + z5 * 128, 128)])
        plsc.subcore_barrier()

        def idx_compute(j, slot):
            a = sidx_v[j, pl.ds(0, 16)] * 4 + p
            b = sidx_v[j, pl.ds(16, 16)] * 4 + p
            ibuf[slot, pl.ds(0, 16)] = a
            ibuf[slot, pl.ds(16, 16)] = b

        def gdesc(j, slot):
            return (
                pltpu.make_async_copy(xl4_hbm.at[ibuf.at[slot]],
                                      gbuf.at[slot], g_sem.at[slot]),
                pltpu.make_async_copy(
                    w_hbm.at[pl.ds((base + j * BE) * H1, BE * H1)],
                    wv.at[slot], w_sem.at[slot]),
            )

        def sstart(j, slot):
            pltpu.async_copy(cbuf.at[slot], acc.at[didx_v.at[j]],
                             s_sem.at[slot], add=True)

        def swait(j, slot):
            pltpu.make_async_copy(cbuf.at[slot], acc.at[didx_v.at[j]],
                                  s_sem.at[slot]).wait()

        for b in range(2):
            idx_compute(b, b)
            for dsc in gdesc(b, b):
                dsc.start()

        def compute_batch(j, slot):
            @pl.when(j >= 2)
            def _():
                swait(j - 2, slot)

            for dsc in gdesc(j, slot):
                dsc.wait()

            def ebody(ep, _):
                wpair = wv[slot, pl.ds(ep * 16, 16)]
                for ei in range(2):
                    e = 2 * ep + ei
                    w0 = wpair[ei * H1 + 2 * p]
                    w1 = wpair[ei * H1 + 2 * p + 1]
                    for c8 in range(8):
                        sl = pl.ds(c8 * 16, 16)
                        ws = w0 if c8 < 4 else w1
                        cbuf[slot, e, sl] = gbuf[slot, e, sl] * ws
                return 0

            lax.fori_loop(0, BE // 2, ebody, 0)
            sstart(j, slot)

            @pl.when(j + 2 < NBATCH)
            def _():
                idx_compute(j + 2, slot)
                for dsc in gdesc(j + 2, slot):
                    dsc.start()

        def outer(i, _):
            for b in range(2):
                compute_batch(2 * i + b, b)
            return 0

        lax.fori_loop(0, NBATCH // 2, outer, 0)
        for b in range(2):
            swait(NBATCH - 2 + b, b)
        plsc.subcore_barrier()
        pltpu.sync_copy(acc.at[pl.ds(tid * _RPT, _RPT)],
                        out4_hbm.at[p, cid, pl.ds(tid * _RPT, _RPT)])


def _sc_scat1(xl4, w, s2, d2):
    mesh = plsc.VectorSubcoreMesh(core_axis_name="c", subcore_axis_name="s")
    f = pl.kernel(
        _sc_scat1_body,
        out_type=jax.ShapeDtypeStruct((4, 2, _NPAD, _RW), jnp.float32),
        mesh=mesh,
        compiler_params=pltpu.CompilerParams(use_tc_tiling_on_sc=False),
        scratch_types=[
            pltpu.VMEM((NBATCH, BE), jnp.int32),
            pltpu.VMEM((NBATCH, BE), jnp.int32),
            pltpu.VMEM((2, BE), jnp.int32),
            pltpu.VMEM((2, BE, 128), jnp.float32),
            pltpu.VMEM((2, BE * H1), jnp.float32),
            pltpu.VMEM((2, BE, _RW), jnp.float32),
            pltpu.VMEM((128, _RW), jnp.float32),
            pltpu.VMEM_SHARED((_NPAD, _RW), jnp.float32),
            pltpu.SemaphoreType.DMA((2,)),
            pltpu.SemaphoreType.DMA((2,)),
            pltpu.SemaphoreType.DMA((2,)),
        ],
    )
    return f(xl4, w, s2, d2)


_RW2 = 80


def _sc_l2_body(xl2_hbm, xr2_hbm, s2_hbm, d2_hbm, att_hbm, out2_hbm,
                sidx_v, didx_v, att_v, glb, grb, cbuf, zbuf, acc,
                gl_sem, gr_sem, s_sem):
    cid = lax.axis_index("c")
    tid = lax.axis_index("s")
    wid = tid * 2 + cid
    base = wid * EPW
    rowb = wid * NBATCH
    lane = lax.iota(jnp.int32, 16)

    pltpu.sync_copy(s2_hbm.at[pl.ds(rowb, NBATCH)], sidx_v)
    pltpu.sync_copy(d2_hbm.at[pl.ds(rowb, NBATCH)], didx_v)
    pltpu.sync_copy(att_hbm, att_v)
    att_regs = [att_v[pl.ds(k * 16, 16)] for k in range(4)]
    perms = [lane ^ sh for sh in (8, 4, 2, 1)]

    def hsum(v):
        for pm in perms:
            v = v + v.at[pm].get(mode="promise_in_bounds")
        return v

    def zrow(r, _):
        for c in range(_RW2 // 16):
            zbuf[r, pl.ds(c * 16, 16)] = jnp.zeros((16,), jnp.float32)
        return 0

    lax.fori_loop(0, 128, zrow, 0)
    for z5 in range(5):
        pltpu.sync_copy(zbuf, acc.at[pl.ds(tid * _RPT + z5 * 128, 128)])
    plsc.subcore_barrier()

    def gdesc(j, slot):
        return (
            pltpu.make_async_copy(xl2_hbm.at[sidx_v.at[j]], glb.at[slot],
                                  gl_sem.at[slot]),
            pltpu.make_async_copy(xr2_hbm.at[didx_v.at[j]], grb.at[slot],
                                  gr_sem.at[slot]),
        )

    def sstart(j, slot):
        pltpu.async_copy(cbuf.at[slot], acc.at[didx_v.at[j]],
                         s_sem.at[slot], add=True)

    def swait(j, slot):
        pltpu.make_async_copy(cbuf.at[slot], acc.at[didx_v.at[j]],
                              s_sem.at[slot]).wait()

    for b in range(2):
        for dsc in gdesc(b, b):
            dsc.start()

    def compute_batch(j, slot):
        @pl.when(j >= 2)
        def _():
            swait(j - 2, slot)

        for dsc in gdesc(j, slot):
            dsc.wait()

        def ebody(e, _):
            accv = jnp.zeros((16,), jnp.float32)
            for c4 in range(4):
                sl = pl.ds(c4 * 16, 16)
                z = glb[slot, e, sl] + grb[slot, e, sl]
                m = jnp.maximum(z, 0.2 * z)
                accv = accv + m * att_regs[c4]
            eidv = lane * 0 + (base + j * BE + e)
            w = jnp.exp(hsum(accv)) * jnp.where(eidv < EALL, 1.0, 0.0)
            for c4 in range(4):
                sl = pl.ds(c4 * 16, 16)
                cbuf[slot, e, sl] = glb[slot, e, sl] * w
            cbuf[slot, e, pl.ds(64, 16)] = jnp.where(lane == 0, w, 0.0)
            return 0

        lax.fori_loop(0, BE, ebody, 0)
        sstart(j, slot)

        @pl.when(j + 2 < NBATCH)
        def _():
            for dsc in gdesc(j + 2, slot):
                dsc.start()

    def outer(i, _):
        for b in range(2):
            compute_batch(2 * i + b, b)
        return 0

    lax.fori_loop(0, NBATCH // 2, outer, 0)
    for b in range(2):
        swait(NBATCH - 2 + b, b)
    plsc.subcore_barrier()
    pltpu.sync_copy(acc.at[pl.ds(tid * _RPT, _RPT)],
                    out2_hbm.at[cid, pl.ds(tid * _RPT, _RPT)])


def _sc_l2(xl2, xr2, s2, d2, att_flat):
    mesh = plsc.VectorSubcoreMesh(core_axis_name="c", subcore_axis_name="s")
    f = pl.kernel(
        _sc_l2_body,
        out_type=jax.ShapeDtypeStruct((2, _NPAD, _RW2), jnp.float32),
        mesh=mesh,
        compiler_params=pltpu.CompilerParams(use_tc_tiling_on_sc=False),
        scratch_types=[
            pltpu.VMEM((NBATCH, BE), jnp.int32),
            pltpu.VMEM((NBATCH, BE), jnp.int32),
            pltpu.VMEM((DH,), jnp.float32),
            pltpu.VMEM((2, BE, DH), jnp.float32),
            pltpu.VMEM((2, BE, DH), jnp.float32),
            pltpu.VMEM((2, BE, _RW2), jnp.float32),
            pltpu.VMEM((128, _RW2), jnp.float32),
            pltpu.VMEM_SHARED((_NPAD, _RW2), jnp.float32),
            pltpu.SemaphoreType.DMA((2,)),
            pltpu.SemaphoreType.DMA((2,)),
            pltpu.SemaphoreType.DMA((2,)),
        ],
    )
    return f(xl2, xr2, s2, d2, att_flat)


def _sc_agg_body(h2_hbm, s2_hbm, d2_hbm, out_hbm,
                 sidx_v, didx_v, glb, cbuf, zbuf, acc,
                 g_sem, s_sem):
    cid = lax.axis_index("c")
    tid = lax.axis_index("s")
    wid = tid * 2 + cid
    base = wid * EPW
    rowb = wid * NBATCH
    lane = lax.iota(jnp.int32, 16)

    pltpu.sync_copy(s2_hbm.at[pl.ds(rowb, NBATCH)], sidx_v)
    pltpu.sync_copy(d2_hbm.at[pl.ds(rowb, NBATCH)], didx_v)

    def zrow(r, _):
        for c in range(4):
            zbuf[r, pl.ds(c * 16, 16)] = jnp.zeros((16,), jnp.float32)
        return 0

    lax.fori_loop(0, 128, zrow, 0)
    for z5 in range(5):
        pltpu.sync_copy(zbuf, acc.at[pl.ds(tid * _RPT + z5 * 128, 128)])
    plsc.subcore_barrier()

    def gdesc(j, slot):
        return pltpu.make_async_copy(h2_hbm.at[sidx_v.at[j]], glb.at[slot],
                                     g_sem.at[slot])

    def sstart(j, slot):
        pltpu.async_copy(cbuf.at[slot], acc.at[didx_v.at[j]],
                         s_sem.at[slot], add=True)

    def swait(j, slot):
        pltpu.make_async_copy(cbuf.at[slot], acc.at[didx_v.at[j]],
                              s_sem.at[slot]).wait()

    for b in range(2):
        gdesc(b, b).start()

    def compute_batch(j, slot):
        @pl.when(j >= 2)
        def _():
            swait(j - 2, slot)

        gdesc(j, slot).wait()

        def ebody(e, _):
            eidv = lane * 0 + (base + j * BE + e)
            msk = jnp.where(eidv < E, 1.0, 0.0)
            for c4 in range(4):
                sl = pl.ds(c4 * 16, 16)
                cbuf[slot, e, sl] = glb[slot, e, sl] * msk
            return 0

        lax.fori_loop(0, BE, ebody, 0)
        sstart(j, slot)

        @pl.when(j + 2 < NBATCH)
        def _():
            gdesc(j + 2, slot).start()

    def outer(i, _):
        for b in range(2):
            compute_batch(2 * i + b, b)
        return 0

    lax.fori_loop(0, NBATCH // 2, outer, 0)
    for b in range(2):
        swait(NBATCH - 2 + b, b)
    plsc.subcore_barrier()
    pltpu.sync_copy(acc.at[pl.ds(tid * _RPT, _RPT)],
                    out_hbm.at[cid, pl.ds(tid * _RPT, _RPT)])


def _sc_agg(h2, s2, d2):
    mesh = plsc.VectorSubcoreMesh(core_axis_name="c", subcore_axis_name="s")
    f = pl.kernel(
        _sc_agg_body,
        out_type=jax.ShapeDtypeStruct((2, _NPAD, DH), jnp.float32),
        mesh=mesh,
        compiler_params=pltpu.CompilerParams(use_tc_tiling_on_sc=False),
        scratch_types=[
            pltpu.VMEM((NBATCH, BE), jnp.int32),
            pltpu.VMEM((NBATCH, BE), jnp.int32),
            pltpu.VMEM((2, BE, DH), jnp.float32),
            pltpu.VMEM((2, BE, DH), jnp.float32),
            pltpu.VMEM((128, DH), jnp.float32),
            pltpu.VMEM_SHARED((_NPAD, DH), jnp.float32),
            pltpu.SemaphoreType.DMA((2,)),
            pltpu.SemaphoreType.DMA((2,)),
        ],
    )
    return f(h2, s2, d2)


def _sortable_i32(f):
    i = lax.bitcast_convert_type(f, jnp.int32)
    return jnp.where(i >= 0, i, i ^ jnp.int32(0x7FFFFFFF))


def _topk_mask(score, batch, counts):
    k = jnp.ceil(RATIO * counts.astype(jnp.float32)).astype(jnp.int32)
    kb = _sortable_i32(score)
    lo = jnp.full((NG,), jnp.iinfo(jnp.int32).min, jnp.int32)
    hi = jnp.full((NG,), jnp.iinfo(jnp.int32).max, jnp.int32)

    def body(_, lohi):
        lo, hi = lohi
        mid = lo + ((hi - lo) >> 1)
        cnt = jax.ops.segment_sum(
            (kb > mid[batch]).astype(jnp.int32), batch, num_segments=NG)
        smaller = cnt < k
        hi = jnp.where(smaller, mid, hi)
        lo = jnp.where(smaller, lo, mid + 1)
        return lo, hi

    lo, hi = lax.fori_loop(0, 32, body, (lo, hi))
    v = lo
    n_greater = jax.ops.segment_sum(
        (kb > v[batch]).astype(jnp.int32), batch, num_segments=NG)
    need = k - n_greater
    tie = (kb == v[batch]).astype(jnp.int32)
    cs = jnp.cumsum(tie)
    tie_per_g = jax.ops.segment_sum(tie, batch, num_segments=NG)
    excl = jnp.cumsum(tie_per_g) - tie_per_g
    tie_rank = cs - excl[batch]
    kept = (kb > v[batch]) | ((tie == 1) & (tie_rank <= need[batch]))
    return kept


def kernel(x, edge_index, batch, Wl1, bl1, Wr1, br1, att1, bias1,
           Wl2, bl2, Wr2, br2, att2, bias2, Wrel, brel, Wroot, Wlin, blin):
    src, dst = edge_index[0], edge_index[1]
    loop = jnp.arange(N, dtype=src.dtype)
    s = jnp.concatenate([src, loop])
    d = jnp.concatenate([dst, loop])
    pad = jnp.zeros((EP - EALL,), jnp.int32)
    s2 = jnp.concatenate([s, pad]).reshape(EP // BE, BE)
    d2 = jnp.concatenate([d, pad]).reshape(EP // BE, BE)

    xl = _proj(x, Wl1, bl1)
    xr = _proj(x, Wr1, br1)
    w1 = _sc_attn1(xl, xr, s2, d2, att1.reshape(_D1))
    out4 = _sc_scat1(xl.reshape(N * 4, 128), w1, s2, d2)
    comb = (out4[:, 0] + out4[:, 1])[:, :N]
    num = comb.transpose(1, 0, 2).reshape(N, H1, DH)
    ex = w1.reshape(EP, H1)[:EALL]
    den = jax.ops.segment_sum(ex, d, num_segments=N)
    out = num / (den[:, :, None] + 1e-16)
    h = jax.nn.elu(out.reshape(N, _D1) + bias1)

    xl2 = _proj(h, Wl2, bl2, blk_rows=1000)
    xr2 = _proj(h, Wr2, br2, blk_rows=1000)
    out2 = _sc_l2(xl2, xr2, s2, d2, att2.reshape(DH))
    comb2 = (out2[0] + out2[1])[:N]
    num2 = comb2[:, :DH]
    den2 = comb2[:, DH]
    h = jax.nn.selu(num2 / (den2[:, None] + 1e-16) + bias2)

    aggp = _sc_agg(h, s2, d2)
    agg = (aggp[0] + aggp[1])[:N]
    score = jnp.tanh((agg @ Wrel + brel + h @ Wroot).reshape(-1))

    counts = jax.ops.segment_sum(jnp.ones((N,), jnp.int32), batch, num_segments=NG)
    kept = _topk_mask(score, batch, counts)

    xo = h * score[:, None]
    xp = jnp.where(kept[:, None], xo, 0.0)
    xm = jnp.where(kept[:, None], xo, -jnp.inf)
    maxp = jax.ops.segment_max(xm, batch, num_segments=NG)
    maxp = jnp.where(jnp.isfinite(maxp), maxp, 0.0)
    sump = jax.ops.segment_sum(xp, batch, num_segments=NG)
    cnt = jax.ops.segment_sum(kept.astype(jnp.float32), batch, num_segments=NG)
    meanp = sump / jnp.clip(cnt, 1.0, None)[:, None]
    feat = jnp.concatenate([maxp, meanp], axis=1)
    out = jax.nn.relu(feat @ Wlin + blin)
    return jax.nn.log_softmax(out, axis=-1)


# speedup vs baseline: 7.1560x; 1.1674x over previous
import functools
import jax
import jax.numpy as jnp
from jax import lax
from jax.experimental import pallas as pl
from jax.experimental.pallas import tpu as pltpu
from jax.experimental.pallas import tpu_sc as plsc

N = 10000
E = 160000
DIN = 128
DH = 64
H1 = 8
NG = 64
RATIO = 0.5

EALL = E + N
NW = 32
BE = 32
NBATCH = 168
EPW = BE * NBATCH
EP = NW * EPW

_D1 = H1 * DH


def _matmul_kernel(x_ref, w_ref, b_ref, o_ref):
    o_ref[...] = (
        jnp.dot(x_ref[...], w_ref[...], preferred_element_type=jnp.float32)
        + b_ref[...]
    )


def _proj(x, W, b, blk_rows=1000):
    n, din = x.shape
    dout = W.shape[1]
    return pl.pallas_call(
        _matmul_kernel,
        out_shape=jax.ShapeDtypeStruct((n, dout), jnp.float32),
        grid=(n // blk_rows,),
        in_specs=[
            pl.BlockSpec((blk_rows, din), lambda i: (i, 0)),
            pl.BlockSpec((din, dout), lambda i: (0, 0)),
            pl.BlockSpec((1, dout), lambda i: (0, 0)),
        ],
        out_specs=pl.BlockSpec((blk_rows, dout), lambda i: (i, 0)),
    )(x, W, b.reshape(1, dout))


def _sc_attn1_body(xl_hbm, xr_hbm, s2_hbm, d2_hbm, att_hbm, w_hbm, den_hbm,
                   sidx_v, didx_v, att_v, glb, grb, wbuf, dbuf, zbuf, dacc,
                   gl_sem, gr_sem, w_sem, d_sem):
    cid = lax.axis_index("c")
    tid = lax.axis_index("s")
    wid = tid * 2 + cid
    base = wid * EPW
    rowb = wid * NBATCH

    def zrow(r, _):
        zbuf[r] = jnp.zeros((16,), jnp.float32)
        return 0

    lax.fori_loop(0, 320, zrow, 0)
    for z2 in range(2):
        pltpu.sync_copy(zbuf, dacc.at[pl.ds(tid * _RPT + z2 * 320, 320)])
    plsc.subcore_barrier()

    pltpu.sync_copy(s2_hbm.at[pl.ds(rowb, NBATCH)], sidx_v)
    pltpu.sync_copy(d2_hbm.at[pl.ds(rowb, NBATCH)], didx_v)
    pltpu.sync_copy(att_hbm, att_v)

    lane = lax.iota(jnp.int32, 16)
    att_regs = [att_v[pl.ds(k * 16, 16)] for k in range(_D1 // 16)]

    def gdesc(j, slot):
        return (
            pltpu.make_async_copy(xl_hbm.at[sidx_v.at[j]], glb.at[slot],
                                  gl_sem.at[slot]),
            pltpu.make_async_copy(xr_hbm.at[didx_v.at[j]], grb.at[slot],
                                  gr_sem.at[slot]),
        )

    def wdesc(j, slot):
        return pltpu.make_async_copy(
            wbuf.at[slot], w_hbm.at[pl.ds((base + j * BE) * H1, BE * H1)],
            w_sem.at[slot])

    def dstart(j, slot):
        pltpu.async_copy(dbuf.at[slot], dacc.at[didx_v.at[j]],
                         d_sem.at[slot], add=True)

    def dwait(j, slot):
        pltpu.make_async_copy(dbuf.at[slot], dacc.at[didx_v.at[j]],
                              d_sem.at[slot]).wait()

    for b in range(2):
        for dsc in gdesc(b, b):
            dsc.start()

    def compute_batch(j, slot):
        for dsc in gdesc(j, slot):
            dsc.wait()

        @pl.when(j >= 2)
        def _():
            wdesc(j - 2, slot).wait()
            dwait(j - 2, slot)

        perms = [lane ^ sh for sh in (8, 4, 2, 1)]

        def hsum(v):
            for pm in perms:
                v = v + v.at[pm].get(mode="promise_in_bounds")
            return v

        def pair_body(p, _):
            e0 = 2 * p
            alpha = jnp.zeros((16,), jnp.float32)
            for ei in range(2):
                for h in range(H1):
                    acc = jnp.zeros((16,), jnp.float32)
                    for c4 in range(4):
                        k = h * 4 + c4
                        sl = pl.ds(k * 16, 16)
                        z = glb[slot, e0 + ei, sl] + grb[slot, e0 + ei, sl]
                        m = jnp.maximum(z, 0.2 * z)
                        acc = acc + m * att_regs[k]
                    alpha = jnp.where(lane == ei * 8 + h, hsum(acc), alpha)
            w16 = jnp.exp(alpha)
            eid = base + j * BE + e0 + (lane >> 3)
            w16 = jnp.where(eid < EALL, w16, 0.0)
            wbuf[slot, pl.ds(p * 16, 16)] = w16
            wsw = w16.at[lane ^ 8].get(mode="promise_in_bounds")
            dbuf[slot, 2 * p] = jnp.where(lane < 8, w16, 0.0)
            dbuf[slot, 2 * p + 1] = jnp.where(lane < 8, wsw, 0.0)
            return 0

        lax.fori_loop(0, BE // 2, pair_body, 0)
        wdesc(j, slot).start()
        dstart(j, slot)

        @pl.when(j + 2 < NBATCH)
        def _():
            for dsc in gdesc(j + 2, slot):
                dsc.start()

    def outer(i, _):
        for b in range(2):
            compute_batch(2 * i + b, b)
        return 0

    lax.fori_loop(0, NBATCH // 2, outer, 0)
    for b in range(2):
        wdesc(NBATCH - 2 + b, b).wait()
        dwait(NBATCH - 2 + b, b)
    plsc.subcore_barrier()
    pltpu.sync_copy(dacc.at[pl.ds(tid * _RPT, _RPT)],
                    den_hbm.at[cid, pl.ds(tid * _RPT, _RPT)])


def _sc_attn1(xl, xr, s2, d2, att_flat):
    mesh = plsc.VectorSubcoreMesh(core_axis_name="c", subcore_axis_name="s")
    f = pl.kernel(
        _sc_attn1_body,
        out_type=(jax.ShapeDtypeStruct((EP * H1,), jnp.float32),
                  jax.ShapeDtypeStruct((2, _NPAD, 16), jnp.float32)),
        mesh=mesh,
        compiler_params=pltpu.CompilerParams(use_tc_tiling_on_sc=False),
        scratch_types=[
            pltpu.VMEM((NBATCH, BE), jnp.int32),
            pltpu.VMEM((NBATCH, BE), jnp.int32),
            pltpu.VMEM((_D1,), jnp.float32),
            pltpu.VMEM((2, BE, _D1), jnp.float32),
            pltpu.VMEM((2, BE, _D1), jnp.float32),
            pltpu.VMEM((2, BE * H1), jnp.float32),
            pltpu.VMEM((2, BE, 16), jnp.float32),
            pltpu.VMEM((320, 16), jnp.float32),
            pltpu.VMEM_SHARED((_NPAD, 16), jnp.float32),
            pltpu.SemaphoreType.DMA((2,)),
            pltpu.SemaphoreType.DMA((2,)),
            pltpu.SemaphoreType.DMA((2,)),
            pltpu.SemaphoreType.DMA((2,)),
        ],
    )
    return f(xl, xr, s2, d2, att_flat)


_RW = 128
_NPAD = 10240
_RPT = _NPAD // 16


def _sc_scat1_body(xl4_hbm, w_hbm, s2_hbm, d2_hbm, out4_hbm,
                   sidx_v, didx_v, ibuf, gbuf, wv, cbuf, zbuf, acc,
                   g_sem, w_sem, s_sem):
    cid = lax.axis_index("c")
    tid = lax.axis_index("s")
    wid = tid * 2 + cid
    base = wid * EPW
    rowb = wid * NBATCH
    lane = lax.iota(jnp.int32, 16)

    pltpu.sync_copy(s2_hbm.at[pl.ds(rowb, NBATCH)], sidx_v)
    pltpu.sync_copy(d2_hbm.at[pl.ds(rowb, NBATCH)], didx_v)

    def zrow(r, _):
        for c9 in range(_RW // 16):
            zbuf[r, pl.ds(c9 * 16, 16)] = jnp.zeros((16,), jnp.float32)
        return 0

    lax.fori_loop(0, 128, zrow, 0)

    for p in range(4):
        for z5 in range(5):
            pltpu.sync_copy(zbuf, acc.at[pl.ds(tid * _RPT + z5 * 128, 128)])
        plsc.subcore_barrier()

        def idx_compute(j, slot):
            a = sidx_v[j, pl.ds(0, 16)] * 4 + p
            b = sidx_v[j, pl.ds(16, 16)] * 4 + p
            ibuf[slot, pl.ds(0, 16)] = a
            ibuf[slot, pl.ds(16, 16)] = b

        def gdesc(j, slot):
            return (
                pltpu.make_async_copy(xl4_hbm.at[ibuf.at[slot]],
                                      gbuf.at[slot], g_sem.at[slot]),
                pltpu.make_async_copy(
                    w_hbm.at[pl.ds((base + j * BE) * H1, BE * H1)],
                    wv.at[slot], w_sem.at[slot]),
            )

        def sstart(j, slot):
            pltpu.async_copy(cbuf.at[slot], acc.at[didx_v.at[j]],
                             s_sem.at[slot], add=True)

        def swait(j, slot):
            pltpu.make_async_copy(cbuf.at[slot], acc.at[didx_v.at[j]],
                                  s_sem.at[slot]).wait()

        for b in range(2):
            idx_compute(b, b)
            for dsc in gdesc(b, b):
                dsc.start()

        def compute_batch(j, slot):
            @pl.when(j >= 2)
            def _():
                swait(j - 2, slot)

            for dsc in gdesc(j, slot):
                dsc.wait()

            def ebody(ep, _):
                wpair = wv[slot, pl.ds(ep * 16, 16)]
                for ei in range(2):
                    e = 2 * ep + ei
                    w0 = wpair[ei * H1 + 2 * p]
                    w1 = wpair[ei * H1 + 2 * p + 1]
                    for c8 in range(8):
                        sl = pl.ds(c8 * 16, 16)
                        ws = w0 if c8 < 4 else w1
                        cbuf[slot, e, sl] = gbuf[slot, e, sl] * ws
                return 0

            lax.fori_loop(0, BE // 2, ebody, 0)
            sstart(j, slot)

            @pl.when(j + 2 < NBATCH)
            def _():
                idx_compute(j + 2, slot)
                for dsc in gdesc(j + 2, slot):
                    dsc.start()

        def outer(i, _):
            for b in range(2):
                compute_batch(2 * i + b, b)
            return 0

        lax.fori_loop(0, NBATCH // 2, outer, 0)
        for b in range(2):
            swait(NBATCH - 2 + b, b)
        plsc.subcore_barrier()
        pltpu.sync_copy(acc.at[pl.ds(tid * _RPT, _RPT)],
                        out4_hbm.at[p, cid, pl.ds(tid * _RPT, _RPT)])


def _sc_scat1(xl4, w, s2, d2):
    mesh = plsc.VectorSubcoreMesh(core_axis_name="c", subcore_axis_name="s")
    f = pl.kernel(
        _sc_scat1_body,
        out_type=jax.ShapeDtypeStruct((4, 2, _NPAD, _RW), jnp.float32),
        mesh=mesh,
        compiler_params=pltpu.CompilerParams(use_tc_tiling_on_sc=False),
        scratch_types=[
            pltpu.VMEM((NBATCH, BE), jnp.int32),
            pltpu.VMEM((NBATCH, BE), jnp.int32),
            pltpu.VMEM((2, BE), jnp.int32),
            pltpu.VMEM((2, BE, 128), jnp.float32),
            pltpu.VMEM((2, BE * H1), jnp.float32),
            pltpu.VMEM((2, BE, _RW), jnp.float32),
            pltpu.VMEM((128, _RW), jnp.float32),
            pltpu.VMEM_SHARED((_NPAD, _RW), jnp.float32),
            pltpu.SemaphoreType.DMA((2,)),
            pltpu.SemaphoreType.DMA((2,)),
            pltpu.SemaphoreType.DMA((2,)),
        ],
    )
    return f(xl4, w, s2, d2)


_RW2 = 80


def _sc_l2_body(xl2_hbm, xr2_hbm, s2_hbm, d2_hbm, att_hbm, out2_hbm,
                sidx_v, didx_v, att_v, glb, grb, cbuf, zbuf, acc,
                gl_sem, gr_sem, s_sem):
    cid = lax.axis_index("c")
    tid = lax.axis_index("s")
    wid = tid * 2 + cid
    base = wid * EPW
    rowb = wid * NBATCH
    lane = lax.iota(jnp.int32, 16)

    pltpu.sync_copy(s2_hbm.at[pl.ds(rowb, NBATCH)], sidx_v)
    pltpu.sync_copy(d2_hbm.at[pl.ds(rowb, NBATCH)], didx_v)
    pltpu.sync_copy(att_hbm, att_v)
    att_regs = [att_v[pl.ds(k * 16, 16)] for k in range(4)]
    perms = [lane ^ sh for sh in (8, 4, 2, 1)]

    def hsum(v):
        for pm in perms:
            v = v + v.at[pm].get(mode="promise_in_bounds")
        return v

    def zrow(r, _):
        for c in range(_RW2 // 16):
            zbuf[r, pl.ds(c * 16, 16)] = jnp.zeros((16,), jnp.float32)
        return 0

    lax.fori_loop(0, 128, zrow, 0)
    for z5 in range(5):
        pltpu.sync_copy(zbuf, acc.at[pl.ds(tid * _RPT + z5 * 128, 128)])
    plsc.subcore_barrier()

    def gdesc(j, slot):
        return (
            pltpu.make_async_copy(xl2_hbm.at[sidx_v.at[j]], glb.at[slot],
                                  gl_sem.at[slot]),
            pltpu.make_async_copy(xr2_hbm.at[didx_v.at[j]], grb.at[slot],
                                  gr_sem.at[slot]),
        )

    def sstart(j, slot):
        pltpu.async_copy(cbuf.at[slot], acc.at[didx_v.at[j]],
                         s_sem.at[slot], add=True)

    def swait(j, slot):
        pltpu.make_async_copy(cbuf.at[slot], acc.at[didx_v.at[j]],
                              s_sem.at[slot]).wait()

    for b in range(2):
        for dsc in gdesc(b, b):
            dsc.start()

    def compute_batch(j, slot):
        @pl.when(j >= 2)
        def _():
            swait(j - 2, slot)

        for dsc in gdesc(j, slot):
            dsc.wait()

        def ebody(e, _):
            accv = jnp.zeros((16,), jnp.float32)
            for c4 in range(4):
                sl = pl.ds(c4 * 16, 16)
                z = glb[slot, e, sl] + grb[slot, e, sl]
                m = jnp.maximum(z, 0.2 * z)
                accv = accv + m * att_regs[c4]
            eidv = lane * 0 + (base + j * BE + e)
            w = jnp.exp(hsum(accv)) * jnp.where(eidv < EALL, 1.0, 0.0)
            for c4 in range(4):
                sl = pl.ds(c4 * 16, 16)
                cbuf[slot, e, sl] = glb[slot, e, sl] * w
            cbuf[slot, e, pl.ds(64, 16)] = jnp.where(lane == 0, w, 0.0)
            return 0

        lax.fori_loop(0, BE, ebody, 0)
        sstart(j, slot)

        @pl.when(j + 2 < NBATCH)
        def _():
            for dsc in gdesc(j + 2, slot):
                dsc.start()

    def outer(i, _):
        for b in range(2):
            compute_batch(2 * i + b, b)
        return 0

    lax.fori_loop(0, NBATCH // 2, outer, 0)
    for b in range(2):
        swait(NBATCH - 2 + b, b)
    plsc.subcore_barrier()
    pltpu.sync_copy(acc.at[pl.ds(tid * _RPT, _RPT)],
                    out2_hbm.at[cid, pl.ds(tid * _RPT, _RPT)])


def _sc_l2(xl2, xr2, s2, d2, att_flat):
    mesh = plsc.VectorSubcoreMesh(core_axis_name="c", subcore_axis_name="s")
    f = pl.kernel(
        _sc_l2_body,
        out_type=jax.ShapeDtypeStruct((2, _NPAD, _RW2), jnp.float32),
        mesh=mesh,
        compiler_params=pltpu.CompilerParams(use_tc_tiling_on_sc=False),
        scratch_types=[
            pltpu.VMEM((NBATCH, BE), jnp.int32),
            pltpu.VMEM((NBATCH, BE), jnp.int32),
            pltpu.VMEM((DH,), jnp.float32),
            pltpu.VMEM((2, BE, DH), jnp.float32),
            pltpu.VMEM((2, BE, DH), jnp.float32),
            pltpu.VMEM((2, BE, _RW2), jnp.float32),
            pltpu.VMEM((128, _RW2), jnp.float32),
            pltpu.VMEM_SHARED((_NPAD, _RW2), jnp.float32),
            pltpu.SemaphoreType.DMA((2,)),
            pltpu.SemaphoreType.DMA((2,)),
            pltpu.SemaphoreType.DMA((2,)),
        ],
    )
    return f(xl2, xr2, s2, d2, att_flat)


def _sc_agg_body(h2_hbm, s2_hbm, d2_hbm, out_hbm,
                 sidx_v, didx_v, glb, cbuf, zbuf, acc,
                 g_sem, s_sem):
    cid = lax.axis_index("c")
    tid = lax.axis_index("s")
    wid = tid * 2 + cid
    base = wid * EPW
    rowb = wid * NBATCH
    lane = lax.iota(jnp.int32, 16)

    pltpu.sync_copy(s2_hbm.at[pl.ds(rowb, NBATCH)], sidx_v)
    pltpu.sync_copy(d2_hbm.at[pl.ds(rowb, NBATCH)], didx_v)

    def zrow(r, _):
        for c in range(4):
            zbuf[r, pl.ds(c * 16, 16)] = jnp.zeros((16,), jnp.float32)
        return 0

    lax.fori_loop(0, 128, zrow, 0)
    for z5 in range(5):
        pltpu.sync_copy(zbuf, acc.at[pl.ds(tid * _RPT + z5 * 128, 128)])
    plsc.subcore_barrier()

    def gdesc(j, slot):
        return pltpu.make_async_copy(h2_hbm.at[sidx_v.at[j]], glb.at[slot],
                                     g_sem.at[slot])

    def sstart(j, slot):
        pltpu.async_copy(cbuf.at[slot], acc.at[didx_v.at[j]],
                         s_sem.at[slot], add=True)

    def swait(j, slot):
        pltpu.make_async_copy(cbuf.at[slot], acc.at[didx_v.at[j]],
                              s_sem.at[slot]).wait()

    for b in range(2):
        gdesc(b, b).start()

    def compute_batch(j, slot):
        @pl.when(j >= 2)
        def _():
            swait(j - 2, slot)

        gdesc(j, slot).wait()

        def ebody(e, _):
            eidv = lane * 0 + (base + j * BE + e)
            msk = jnp.where(eidv < E, 1.0, 0.0)
            for c4 in range(4):
                sl = pl.ds(c4 * 16, 16)
                cbuf[slot, e, sl] = glb[slot, e, sl] * msk
            return 0

        lax.fori_loop(0, BE, ebody, 0)
        sstart(j, slot)

        @pl.when(j + 2 < NBATCH)
        def _():
            gdesc(j + 2, slot).start()

    def outer(i, _):
        for b in range(2):
            compute_batch(2 * i + b, b)
        return 0

    lax.fori_loop(0, NBATCH // 2, outer, 0)
    for b in range(2):
        swait(NBATCH - 2 + b, b)
    plsc.subcore_barrier()
    pltpu.sync_copy(acc.at[pl.ds(tid * _RPT, _RPT)],
                    out_hbm.at[cid, pl.ds(tid * _RPT, _RPT)])


def _sc_agg(h2, s2, d2):
    mesh = plsc.VectorSubcoreMesh(core_axis_name="c", subcore_axis_name="s")
    f = pl.kernel(
        _sc_agg_body,
        out_type=jax.ShapeDtypeStruct((2, _NPAD, DH), jnp.float32),
        mesh=mesh,
        compiler_params=pltpu.CompilerParams(use_tc_tiling_on_sc=False),
        scratch_types=[
            pltpu.VMEM((NBATCH, BE), jnp.int32),
            pltpu.VMEM((NBATCH, BE), jnp.int32),
            pltpu.VMEM((2, BE, DH), jnp.float32),
            pltpu.VMEM((2, BE, DH), jnp.float32),
            pltpu.VMEM((128, DH), jnp.float32),
            pltpu.VMEM_SHARED((_NPAD, DH), jnp.float32),
            pltpu.SemaphoreType.DMA((2,)),
            pltpu.SemaphoreType.DMA((2,)),
        ],
    )
    return f(h2, s2, d2)


def _sortable_i32(f):
    i = lax.bitcast_convert_type(f, jnp.int32)
    return jnp.where(i >= 0, i, i ^ jnp.int32(0x7FFFFFFF))


def _topk_mask(score, batch, counts):
    k = jnp.ceil(RATIO * counts.astype(jnp.float32)).astype(jnp.int32)
    kb = _sortable_i32(score)
    lo = jnp.full((NG,), jnp.iinfo(jnp.int32).min, jnp.int32)
    hi = jnp.full((NG,), jnp.iinfo(jnp.int32).max, jnp.int32)

    def body(_, lohi):
        lo, hi = lohi
        mid = lo + ((hi - lo) >> 1)
        cnt = jax.ops.segment_sum(
            (kb > mid[batch]).astype(jnp.int32), batch, num_segments=NG)
        smaller = cnt < k
        hi = jnp.where(smaller, mid, hi)
        lo = jnp.where(smaller, lo, mid + 1)
        return lo, hi

    lo, hi = lax.fori_loop(0, 32, body, (lo, hi))
    v = lo
    n_greater = jax.ops.segment_sum(
        (kb > v[batch]).astype(jnp.int32), batch, num_segments=NG)
    need = k - n_greater
    tie = (kb == v[batch]).astype(jnp.int32)
    cs = jnp.cumsum(tie)
    tie_per_g = jax.ops.segment_sum(tie, batch, num_segments=NG)
    excl = jnp.cumsum(tie_per_g) - tie_per_g
    tie_rank = cs - excl[batch]
    kept = (kb > v[batch]) | ((tie == 1) & (tie_rank <= need[batch]))
    return kept


def kernel(x, edge_index, batch, Wl1, bl1, Wr1, br1, att1, bias1,
           Wl2, bl2, Wr2, br2, att2, bias2, Wrel, brel, Wroot, Wlin, blin):
    src, dst = edge_index[0], edge_index[1]
    loop = jnp.arange(N, dtype=src.dtype)
    s = jnp.concatenate([src, loop])
    d = jnp.concatenate([dst, loop])
    pad = jnp.zeros((EP - EALL,), jnp.int32)
    s2 = jnp.concatenate([s, pad]).reshape(EP // BE, BE)
    d2 = jnp.concatenate([d, pad]).reshape(EP // BE, BE)

    xl = _proj(x, Wl1, bl1)
    xr = _proj(x, Wr1, br1)
    w1, denp = _sc_attn1(xl, xr, s2, d2, att1.reshape(_D1))
    out4 = _sc_scat1(xl.reshape(N * 4, 128), w1, s2, d2)
    comb = (out4[:, 0] + out4[:, 1])[:, :N]
    num = comb.transpose(1, 0, 2).reshape(N, H1, DH)
    den = (denp[0] + denp[1])[:N, :H1]
    out = num / (den[:, :, None] + 1e-16)
    h = jax.nn.elu(out.reshape(N, _D1) + bias1)

    xl2 = _proj(h, Wl2, bl2, blk_rows=1000)
    xr2 = _proj(h, Wr2, br2, blk_rows=1000)
    out2 = _sc_l2(xl2, xr2, s2, d2, att2.reshape(DH))
    comb2 = (out2[0] + out2[1])[:N]
    num2 = comb2[:, :DH]
    den2 = comb2[:, DH]
    h = jax.nn.selu(num2 / (den2[:, None] + 1e-16) + bias2)

    aggp = _sc_agg(h, s2, d2)
    agg = (aggp[0] + aggp[1])[:N]
    score = jnp.tanh((agg @ Wrel + brel + h @ Wroot).reshape(-1))

    counts = jax.ops.segment_sum(jnp.ones((N,), jnp.int32), batch, num_segments=NG)
    kept = _topk_mask(score, batch, counts)

    xo = h * score[:, None]
    xp = jnp.where(kept[:, None], xo, 0.0)
    xm = jnp.where(kept[:, None], xo, -jnp.inf)
    maxp = jax.ops.segment_max(xm, batch, num_segments=NG)
    maxp = jnp.where(jnp.isfinite(maxp), maxp, 0.0)
    sump = jax.ops.segment_sum(xp, batch, num_segments=NG)
    cnt = jax.ops.segment_sum(kept.astype(jnp.float32), batch, num_segments=NG)
    meanp = sump / jnp.clip(cnt, 1.0, None)[:, None]
    feat = jnp.concatenate([maxp, meanp], axis=1)
    out = jax.nn.relu(feat @ Wlin + blin)
    return jax.nn.log_softmax(out, axis=-1)


# speedup vs baseline: 14.8776x; 2.0790x over previous
import jax
import jax.numpy as jnp
from jax import lax
from jax.experimental import pallas as pl
from jax.experimental.pallas import tpu as pltpu
from jax.experimental.pallas import tpu_sc as plsc

N = 10000
E = 160000
DIN = 128
DH = 64
H1 = 8
NG = 64
RATIO = 0.5

EALL = E + N
NW = 32
BE = 32
NBATCH = 168
EPW = BE * NBATCH
EP = NW * EPW

_D1 = H1 * DH


def _matmul_kernel(x_ref, w_ref, b_ref, o_ref):
    o_ref[...] = (
        jnp.dot(x_ref[...], w_ref[...], preferred_element_type=jnp.float32)
        + b_ref[...]
    )


def _proj(x, W, b, blk_rows=1000):
    n, din = x.shape
    dout = W.shape[1]
    return pl.pallas_call(
        _matmul_kernel,
        out_shape=jax.ShapeDtypeStruct((n, dout), jnp.float32),
        grid=(n // blk_rows,),
        in_specs=[
            pl.BlockSpec((blk_rows, din), lambda i: (i, 0)),
            pl.BlockSpec((din, dout), lambda i: (0, 0)),
            pl.BlockSpec((1, dout), lambda i: (0, 0)),
        ],
        out_specs=pl.BlockSpec((blk_rows, dout), lambda i: (i, 0)),
    )(x, W, b.reshape(1, dout))


def _sc_attn1_body(xl_hbm, xr_hbm, s2_hbm, d2_hbm, att_hbm, w_hbm, den_hbm,
                   sidx_v, didx_v, att_v, glb, grb, wbuf, dbuf, zbuf, dacc,
                   gl_sem, gr_sem, w_sem, d_sem):
    cid = lax.axis_index("c")
    tid = lax.axis_index("s")
    wid = tid * 2 + cid
    base = wid * EPW
    rowb = wid * NBATCH

    def zrow(r, _):
        zbuf[r] = jnp.zeros((16,), jnp.float32)
        return 0

    lax.fori_loop(0, 320, zrow, 0)
    for z2 in range(2):
        pltpu.sync_copy(zbuf, dacc.at[pl.ds(tid * _RPT + z2 * 320, 320)])
    plsc.subcore_barrier()

    pltpu.sync_copy(s2_hbm.at[pl.ds(rowb, NBATCH)], sidx_v)
    pltpu.sync_copy(d2_hbm.at[pl.ds(rowb, NBATCH)], didx_v)
    pltpu.sync_copy(att_hbm, att_v)

    lane = lax.iota(jnp.int32, 16)
    att_regs = [att_v[pl.ds(k * 16, 16)] for k in range(_D1 // 16)]

    def gdesc(j, slot):
        return (
            pltpu.make_async_copy(xl_hbm.at[sidx_v.at[j]], glb.at[slot],
                                  gl_sem.at[slot]),
            pltpu.make_async_copy(xr_hbm.at[didx_v.at[j]], grb.at[slot],
                                  gr_sem.at[slot]),
        )

    def wdesc(j, slot):
        return pltpu.make_async_copy(
            wbuf.at[slot], w_hbm.at[pl.ds((base + j * BE) * H1, BE * H1)],
            w_sem.at[slot])

    def dstart(j, slot):
        pltpu.async_copy(dbuf.at[slot], dacc.at[didx_v.at[j]],
                         d_sem.at[slot], add=True)

    def dwait(j, slot):
        pltpu.make_async_copy(dbuf.at[slot], dacc.at[didx_v.at[j]],
                              d_sem.at[slot]).wait()

    for b in range(2):
        for dsc in gdesc(b, b):
            dsc.start()

    def compute_batch(j, slot):
        for dsc in gdesc(j, slot):
            dsc.wait()

        @pl.when(j >= 2)
        def _():
            wdesc(j - 2, slot).wait()
            dwait(j - 2, slot)

        perms = [lane ^ sh for sh in (8, 4, 2, 1)]

        def hsum(v):
            for pm in perms:
                v = v + v.at[pm].get(mode="promise_in_bounds")
            return v

        def pair_body(p, _):
            e0 = 2 * p
            alpha = jnp.zeros((16,), jnp.float32)
            for ei in range(2):
                for h in range(H1):
                    acc = jnp.zeros((16,), jnp.float32)
                    for c4 in range(4):
                        k = h * 4 + c4
                        sl = pl.ds(k * 16, 16)
                        z = glb[slot, e0 + ei, sl] + grb[slot, e0 + ei, sl]
                        m = jnp.maximum(z, 0.2 * z)
                        acc = acc + m * att_regs[k]
                    alpha = jnp.where(lane == ei * 8 + h, hsum(acc), alpha)
            w16 = jnp.exp(alpha)
            eid = base + j * BE + e0 + (lane >> 3)
            w16 = jnp.where(eid < EALL, w16, 0.0)
            wbuf[slot, pl.ds(p * 16, 16)] = w16
            wsw = w16.at[lane ^ 8].get(mode="promise_in_bounds")
            dbuf[slot, 2 * p] = jnp.where(lane < 8, w16, 0.0)
            dbuf[slot, 2 * p + 1] = jnp.where(lane < 8, wsw, 0.0)
            return 0

        lax.fori_loop(0, BE // 2, pair_body, 0)
        wdesc(j, slot).start()
        dstart(j, slot)

        @pl.when(j + 2 < NBATCH)
        def _():
            for dsc in gdesc(j + 2, slot):
                dsc.start()

    def outer(i, _):
        for b in range(2):
            compute_batch(2 * i + b, b)
        return 0

    lax.fori_loop(0, NBATCH // 2, outer, 0)
    for b in range(2):
        wdesc(NBATCH - 2 + b, b).wait()
        dwait(NBATCH - 2 + b, b)
    plsc.subcore_barrier()
    pltpu.sync_copy(dacc.at[pl.ds(tid * _RPT, _RPT)],
                    den_hbm.at[cid, pl.ds(tid * _RPT, _RPT)])


def _sc_attn1(xl, xr, s2, d2, att_flat):
    mesh = plsc.VectorSubcoreMesh(core_axis_name="c", subcore_axis_name="s")
    f = pl.kernel(
        _sc_attn1_body,
        out_type=(jax.ShapeDtypeStruct((EP * H1,), jnp.float32),
                  jax.ShapeDtypeStruct((2, _NPAD, 16), jnp.float32)),
        mesh=mesh,
        compiler_params=pltpu.CompilerParams(use_tc_tiling_on_sc=False),
        scratch_types=[
            pltpu.VMEM((NBATCH, BE), jnp.int32),
            pltpu.VMEM((NBATCH, BE), jnp.int32),
            pltpu.VMEM((_D1,), jnp.float32),
            pltpu.VMEM((2, BE, _D1), jnp.float32),
            pltpu.VMEM((2, BE, _D1), jnp.float32),
            pltpu.VMEM((2, BE * H1), jnp.float32),
            pltpu.VMEM((2, BE, 16), jnp.float32),
            pltpu.VMEM((320, 16), jnp.float32),
            pltpu.VMEM_SHARED((_NPAD, 16), jnp.float32),
            pltpu.SemaphoreType.DMA((2,)),
            pltpu.SemaphoreType.DMA((2,)),
            pltpu.SemaphoreType.DMA((2,)),
            pltpu.SemaphoreType.DMA((2,)),
        ],
    )
    return f(xl, xr, s2, d2, att_flat)


_RW = 128
_NPAD = 10240
_RPT = _NPAD // 16


def _sc_scat1_body(xl4_hbm, w_hbm, s2_hbm, d2_hbm, out4_hbm,
                   sidx_v, didx_v, ibuf, gbuf, wv, cbuf, zbuf, acc,
                   g_sem, w_sem, s_sem):
    cid = lax.axis_index("c")
    tid = lax.axis_index("s")
    wid = tid * 2 + cid
    base = wid * EPW
    rowb = wid * NBATCH
    lane = lax.iota(jnp.int32, 16)

    pltpu.sync_copy(s2_hbm.at[pl.ds(rowb, NBATCH)], sidx_v)
    pltpu.sync_copy(d2_hbm.at[pl.ds(rowb, NBATCH)], didx_v)

    def zrow(r, _):
        for c9 in range(_RW // 16):
            zbuf[r, pl.ds(c9 * 16, 16)] = jnp.zeros((16,), jnp.float32)
        return 0

    lax.fori_loop(0, 128, zrow, 0)

    for p in range(4):
        for z5 in range(5):
            pltpu.sync_copy(zbuf, acc.at[pl.ds(tid * _RPT + z5 * 128, 128)])
        plsc.subcore_barrier()

        def idx_compute(j, slot):
            a = sidx_v[j, pl.ds(0, 16)] * 4 + p
            b = sidx_v[j, pl.ds(16, 16)] * 4 + p
            ibuf[slot, pl.ds(0, 16)] = a
            ibuf[slot, pl.ds(16, 16)] = b

        def gdesc(j, slot):
            return (
                pltpu.make_async_copy(xl4_hbm.at[ibuf.at[slot]],
                                      gbuf.at[slot], g_sem.at[slot]),
                pltpu.make_async_copy(
                    w_hbm.at[pl.ds((base + j * BE) * H1, BE * H1)],
                    wv.at[slot], w_sem.at[slot]),
            )

        def sstart(j, slot):
            pltpu.async_copy(cbuf.at[slot], acc.at[didx_v.at[j]],
                             s_sem.at[slot], add=True)

        def swait(j, slot):
            pltpu.make_async_copy(cbuf.at[slot], acc.at[didx_v.at[j]],
                                  s_sem.at[slot]).wait()

        for b in range(2):
            idx_compute(b, b)
            for dsc in gdesc(b, b):
                dsc.start()

        def compute_batch(j, slot):
            @pl.when(j >= 2)
            def _():
                swait(j - 2, slot)

            for dsc in gdesc(j, slot):
                dsc.wait()

            def ebody(ep, _):
                wpair = wv[slot, pl.ds(ep * 16, 16)]
                for ei in range(2):
                    e = 2 * ep + ei
                    w0 = wpair[ei * H1 + 2 * p]
                    w1 = wpair[ei * H1 + 2 * p + 1]
                    for c8 in range(8):
                        sl = pl.ds(c8 * 16, 16)
                        ws = w0 if c8 < 4 else w1
                        cbuf[slot, e, sl] = gbuf[slot, e, sl] * ws
                return 0

            lax.fori_loop(0, BE // 2, ebody, 0)
            sstart(j, slot)

            @pl.when(j + 2 < NBATCH)
            def _():
                idx_compute(j + 2, slot)
                for dsc in gdesc(j + 2, slot):
                    dsc.start()

        def outer(i, _):
            for b in range(2):
                compute_batch(2 * i + b, b)
            return 0

        lax.fori_loop(0, NBATCH // 2, outer, 0)
        for b in range(2):
            swait(NBATCH - 2 + b, b)
        plsc.subcore_barrier()
        pltpu.sync_copy(acc.at[pl.ds(tid * _RPT, _RPT)],
                        out4_hbm.at[p, cid, pl.ds(tid * _RPT, _RPT)])


def _sc_scat1(xl4, w, s2, d2):
    mesh = plsc.VectorSubcoreMesh(core_axis_name="c", subcore_axis_name="s")
    f = pl.kernel(
        _sc_scat1_body,
        out_type=jax.ShapeDtypeStruct((4, 2, _NPAD, _RW), jnp.float32),
        mesh=mesh,
        compiler_params=pltpu.CompilerParams(use_tc_tiling_on_sc=False),
        scratch_types=[
            pltpu.VMEM((NBATCH, BE), jnp.int32),
            pltpu.VMEM((NBATCH, BE), jnp.int32),
            pltpu.VMEM((2, BE), jnp.int32),
            pltpu.VMEM((2, BE, 128), jnp.float32),
            pltpu.VMEM((2, BE * H1), jnp.float32),
            pltpu.VMEM((2, BE, _RW), jnp.float32),
            pltpu.VMEM((128, _RW), jnp.float32),
            pltpu.VMEM_SHARED((_NPAD, _RW), jnp.float32),
            pltpu.SemaphoreType.DMA((2,)),
            pltpu.SemaphoreType.DMA((2,)),
            pltpu.SemaphoreType.DMA((2,)),
        ],
    )
    return f(xl4, w, s2, d2)


_RW2 = 80


def _sc_l2_body(xl2_hbm, xr2_hbm, s2_hbm, d2_hbm, att_hbm, out2_hbm,
                sidx_v, didx_v, att_v, glb, grb, cbuf, zbuf, acc,
                gl_sem, gr_sem, s_sem):
    cid = lax.axis_index("c")
    tid = lax.axis_index("s")
    wid = tid * 2 + cid
    base = wid * EPW
    rowb = wid * NBATCH
    lane = lax.iota(jnp.int32, 16)

    pltpu.sync_copy(s2_hbm.at[pl.ds(rowb, NBATCH)], sidx_v)
    pltpu.sync_copy(d2_hbm.at[pl.ds(rowb, NBATCH)], didx_v)
    pltpu.sync_copy(att_hbm, att_v)
    att_regs = [att_v[pl.ds(k * 16, 16)] for k in range(4)]
    perms = [lane ^ sh for sh in (8, 4, 2, 1)]

    def hsum(v):
        for pm in perms:
            v = v + v.at[pm].get(mode="promise_in_bounds")
        return v

    def zrow(r, _):
        for c in range(_RW2 // 16):
            zbuf[r, pl.ds(c * 16, 16)] = jnp.zeros((16,), jnp.float32)
        return 0

    lax.fori_loop(0, 128, zrow, 0)
    for z5 in range(5):
        pltpu.sync_copy(zbuf, acc.at[pl.ds(tid * _RPT + z5 * 128, 128)])
    plsc.subcore_barrier()

    def gdesc(j, slot):
        return (
            pltpu.make_async_copy(xl2_hbm.at[sidx_v.at[j]], glb.at[slot],
                                  gl_sem.at[slot]),
            pltpu.make_async_copy(xr2_hbm.at[didx_v.at[j]], grb.at[slot],
                                  gr_sem.at[slot]),
        )

    def sstart(j, slot):
        pltpu.async_copy(cbuf.at[slot], acc.at[didx_v.at[j]],
                         s_sem.at[slot], add=True)

    def swait(j, slot):
        pltpu.make_async_copy(cbuf.at[slot], acc.at[didx_v.at[j]],
                              s_sem.at[slot]).wait()

    for b in range(2):
        for dsc in gdesc(b, b):
            dsc.start()

    def compute_batch(j, slot):
        @pl.when(j >= 2)
        def _():
            swait(j - 2, slot)

        for dsc in gdesc(j, slot):
            dsc.wait()

        def ebody(e, _):
            accv = jnp.zeros((16,), jnp.float32)
            for c4 in range(4):
                sl = pl.ds(c4 * 16, 16)
                z = glb[slot, e, sl] + grb[slot, e, sl]
                m = jnp.maximum(z, 0.2 * z)
                accv = accv + m * att_regs[c4]
            eidv = lane * 0 + (base + j * BE + e)
            w = jnp.exp(hsum(accv)) * jnp.where(eidv < EALL, 1.0, 0.0)
            for c4 in range(4):
                sl = pl.ds(c4 * 16, 16)
                cbuf[slot, e, sl] = glb[slot, e, sl] * w
            cbuf[slot, e, pl.ds(64, 16)] = jnp.where(lane == 0, w, 0.0)
            return 0

        lax.fori_loop(0, BE, ebody, 0)
        sstart(j, slot)

        @pl.when(j + 2 < NBATCH)
        def _():
            for dsc in gdesc(j + 2, slot):
                dsc.start()

    def outer(i, _):
        for b in range(2):
            compute_batch(2 * i + b, b)
        return 0

    lax.fori_loop(0, NBATCH // 2, outer, 0)
    for b in range(2):
        swait(NBATCH - 2 + b, b)
    plsc.subcore_barrier()
    pltpu.sync_copy(acc.at[pl.ds(tid * _RPT, _RPT)],
                    out2_hbm.at[cid, pl.ds(tid * _RPT, _RPT)])


def _sc_l2(xl2, xr2, s2, d2, att_flat):
    mesh = plsc.VectorSubcoreMesh(core_axis_name="c", subcore_axis_name="s")
    f = pl.kernel(
        _sc_l2_body,
        out_type=jax.ShapeDtypeStruct((2, _NPAD, _RW2), jnp.float32),
        mesh=mesh,
        compiler_params=pltpu.CompilerParams(use_tc_tiling_on_sc=False),
        scratch_types=[
            pltpu.VMEM((NBATCH, BE), jnp.int32),
            pltpu.VMEM((NBATCH, BE), jnp.int32),
            pltpu.VMEM((DH,), jnp.float32),
            pltpu.VMEM((2, BE, DH), jnp.float32),
            pltpu.VMEM((2, BE, DH), jnp.float32),
            pltpu.VMEM((2, BE, _RW2), jnp.float32),
            pltpu.VMEM((128, _RW2), jnp.float32),
            pltpu.VMEM_SHARED((_NPAD, _RW2), jnp.float32),
            pltpu.SemaphoreType.DMA((2,)),
            pltpu.SemaphoreType.DMA((2,)),
            pltpu.SemaphoreType.DMA((2,)),
        ],
    )
    return f(xl2, xr2, s2, d2, att_flat)


def _sc_agg_body(h2_hbm, s2_hbm, d2_hbm, out_hbm,
                 sidx_v, didx_v, glb, cbuf, zbuf, acc,
                 g_sem, s_sem):
    cid = lax.axis_index("c")
    tid = lax.axis_index("s")
    wid = tid * 2 + cid
    base = wid * EPW
    rowb = wid * NBATCH
    lane = lax.iota(jnp.int32, 16)

    pltpu.sync_copy(s2_hbm.at[pl.ds(rowb, NBATCH)], sidx_v)
    pltpu.sync_copy(d2_hbm.at[pl.ds(rowb, NBATCH)], didx_v)

    def zrow(r, _):
        for c in range(4):
            zbuf[r, pl.ds(c * 16, 16)] = jnp.zeros((16,), jnp.float32)
        return 0

    lax.fori_loop(0, 128, zrow, 0)
    for z5 in range(5):
        pltpu.sync_copy(zbuf, acc.at[pl.ds(tid * _RPT + z5 * 128, 128)])
    plsc.subcore_barrier()

    def gdesc(j, slot):
        return pltpu.make_async_copy(h2_hbm.at[sidx_v.at[j]], glb.at[slot],
                                     g_sem.at[slot])

    def sstart(j, slot):
        pltpu.async_copy(cbuf.at[slot], acc.at[didx_v.at[j]],
                         s_sem.at[slot], add=True)

    def swait(j, slot):
        pltpu.make_async_copy(cbuf.at[slot], acc.at[didx_v.at[j]],
                              s_sem.at[slot]).wait()

    for b in range(2):
        gdesc(b, b).start()

    def compute_batch(j, slot):
        @pl.when(j >= 2)
        def _():
            swait(j - 2, slot)

        gdesc(j, slot).wait()

        def ebody(e, _):
            eidv = lane * 0 + (base + j * BE + e)
            msk = jnp.where(eidv < E, 1.0, 0.0)
            for c4 in range(4):
                sl = pl.ds(c4 * 16, 16)
                cbuf[slot, e, sl] = glb[slot, e, sl] * msk
            return 0

        lax.fori_loop(0, BE, ebody, 0)
        sstart(j, slot)

        @pl.when(j + 2 < NBATCH)
        def _():
            gdesc(j + 2, slot).start()

    def outer(i, _):
        for b in range(2):
            compute_batch(2 * i + b, b)
        return 0

    lax.fori_loop(0, NBATCH // 2, outer, 0)
    for b in range(2):
        swait(NBATCH - 2 + b, b)
    plsc.subcore_barrier()
    pltpu.sync_copy(acc.at[pl.ds(tid * _RPT, _RPT)],
                    out_hbm.at[cid, pl.ds(tid * _RPT, _RPT)])


def _sc_agg(h2, s2, d2):
    mesh = plsc.VectorSubcoreMesh(core_axis_name="c", subcore_axis_name="s")
    f = pl.kernel(
        _sc_agg_body,
        out_type=jax.ShapeDtypeStruct((2, _NPAD, DH), jnp.float32),
        mesh=mesh,
        compiler_params=pltpu.CompilerParams(use_tc_tiling_on_sc=False),
        scratch_types=[
            pltpu.VMEM((NBATCH, BE), jnp.int32),
            pltpu.VMEM((NBATCH, BE), jnp.int32),
            pltpu.VMEM((2, BE, DH), jnp.float32),
            pltpu.VMEM((2, BE, DH), jnp.float32),
            pltpu.VMEM((128, DH), jnp.float32),
            pltpu.VMEM_SHARED((_NPAD, DH), jnp.float32),
            pltpu.SemaphoreType.DMA((2,)),
            pltpu.SemaphoreType.DMA((2,)),
        ],
    )
    return f(h2, s2, d2)


def _final_kernel(h2_ref, a_ref, b_ref, wrel_ref, brel_ref,
                  wroot_ref, wlin_ref, blin_ref, o_ref):
    h2 = h2_ref[...]
    agg = a_ref[...]
    score = jnp.tanh(
        jnp.dot(agg, wrel_ref[...], preferred_element_type=jnp.float32)
        + brel_ref[...]
        + jnp.dot(h2, wroot_ref[...], preferred_element_type=jnp.float32))

    batch2 = b_ref[...]
    gidx = lax.broadcasted_iota(jnp.int32, (N, NG), 1)
    Bm = batch2 == gidx
    Bf = jnp.where(Bm, 1.0, 0.0)
    counts = jnp.sum(Bf, axis=0, keepdims=True)
    k = jnp.floor((counts + 1.0) * 0.5)

    bits = lax.bitcast_convert_type(score, jnp.uint32)
    neg = bits >= jnp.uint32(0x80000000)
    key = jnp.where(neg, ~bits, bits | jnp.uint32(0x80000000))
    lo = jnp.zeros((1, NG), jnp.uint32)
    hi = jnp.full((1, NG), jnp.uint32(0xFFFFFFFF), jnp.uint32)

    def sbody(_, lohi):
        lo, hi = lohi
        mid = lo + ((hi - lo) >> jnp.uint32(1))
        cnt = jnp.sum(jnp.where(Bm & (key > mid), 1.0, 0.0),
                      axis=0, keepdims=True)
        smaller = cnt < k
        return (jnp.where(smaller, lo, mid + jnp.uint32(1)),
                jnp.where(smaller, mid, hi))

    lo, hi = lax.fori_loop(0, 32, sbody, (lo, hi))
    v = lo
    CvB = jnp.where(Bm & (key > v), 1.0, 0.0)
    greater = jnp.sum(CvB, axis=1, keepdims=True)
    n_greater = jnp.sum(CvB, axis=0, keepdims=True)
    TvB = jnp.where(Bm & (key == v), 1.0, 0.0)
    tie = jnp.sum(TvB, axis=1, keepdims=True)
    need = k - n_greater

    idxcol = lax.broadcasted_iota(jnp.int32, (N, 1), 0)
    tlo = jnp.zeros((1, NG), jnp.int32)
    thi = jnp.full((1, NG), N - 1, jnp.int32)

    def tbody(_, lohi):
        tlo, thi = lohi
        mid = tlo + ((thi - tlo) >> 1)
        cnt = jnp.sum(TvB * jnp.where(idxcol <= mid, 1.0, 0.0),
                      axis=0, keepdims=True)
        ok = cnt >= need
        return jnp.where(ok, tlo, mid + 1), jnp.where(ok, mid, thi)

    tlo, thi = lax.fori_loop(0, 14, tbody, (tlo, thi))
    t_node = jnp.sum(Bf * tlo.astype(jnp.float32), axis=1, keepdims=True)
    need_node = jnp.sum(Bf * need, axis=1, keepdims=True)
    keptf = jnp.where(
        (greater > 0.0)
        | ((tie > 0.0) & (need_node >= 1.0)
           & (idxcol.astype(jnp.float32) <= t_node)), 1.0, 0.0)

    xo = h2 * score
    BfK = Bf * keptf
    sump = lax.dot_general(BfK, xo, (((0,), (0,)), ((), ())),
                           preferred_element_type=jnp.float32)
    ones_n = jnp.ones((N, 1), jnp.float32)
    cntc = lax.dot_general(BfK, ones_n, (((0,), (0,)), ((), ())),
                           preferred_element_type=jnp.float32)
    meanp = sump / jnp.maximum(cntc, 1.0)

    NEG = jnp.float32(-3.0e38)
    growi = lax.broadcasted_iota(jnp.int32, (NG, DH), 0)

    def gbody(g, maxp):
        colg = (batch2 == g) & (keptf > 0.0)
        m = jnp.where(colg, xo, NEG)
        mx = jnp.max(m, axis=0, keepdims=True)
        return jnp.where(growi == g, mx, maxp)

    maxp = lax.fori_loop(0, NG, gbody, jnp.full((NG, DH), NEG, jnp.float32))
    maxp = jnp.where(cntc > 0.0, maxp, 0.0)

    feat = jnp.concatenate([maxp, meanp], axis=1)
    o = jnp.dot(feat, wlin_ref[...], preferred_element_type=jnp.float32)
    o = jnp.maximum(o + blin_ref[...], 0.0)
    mx = jnp.max(o, axis=1, keepdims=True)
    lse = mx + jnp.log(jnp.sum(jnp.exp(o - mx), axis=1, keepdims=True))
    o_ref[...] = o - lse


def _final(h2, agg, batch, Wrel, brel, Wroot, Wlin, blin):
    return pl.pallas_call(
        _final_kernel,
        out_shape=jax.ShapeDtypeStruct((NG, 3), jnp.float32),
    )(h2, agg, batch.reshape(N, 1), Wrel, brel.reshape(1, 1),
      Wroot, Wlin, blin.reshape(1, 3))


def kernel(x, edge_index, batch, Wl1, bl1, Wr1, br1, att1, bias1,
           Wl2, bl2, Wr2, br2, att2, bias2, Wrel, brel, Wroot, Wlin, blin):
    src, dst = edge_index[0], edge_index[1]
    loop = jnp.arange(N, dtype=src.dtype)
    s = jnp.concatenate([src, loop])
    d = jnp.concatenate([dst, loop])
    pad = jnp.zeros((EP - EALL,), jnp.int32)
    s2 = jnp.concatenate([s, pad]).reshape(EP // BE, BE)
    d2 = jnp.concatenate([d, pad]).reshape(EP // BE, BE)

    xl = _proj(x, Wl1, bl1)
    xr = _proj(x, Wr1, br1)
    w1, denp = _sc_attn1(xl, xr, s2, d2, att1.reshape(_D1))
    out4 = _sc_scat1(xl.reshape(N * 4, 128), w1, s2, d2)
    comb = (out4[:, 0] + out4[:, 1])[:, :N]
    num = comb.transpose(1, 0, 2).reshape(N, H1, DH)
    den = (denp[0] + denp[1])[:N, :H1]
    out = num / (den[:, :, None] + 1e-16)
    h = jax.nn.elu(out.reshape(N, _D1) + bias1)

    xl2 = _proj(h, Wl2, bl2, blk_rows=1000)
    xr2 = _proj(h, Wr2, br2, blk_rows=1000)
    out2 = _sc_l2(xl2, xr2, s2, d2, att2.reshape(DH))
    comb2 = (out2[0] + out2[1])[:N]
    num2 = comb2[:, :DH]
    den2 = comb2[:, DH]
    h = jax.nn.selu(num2 / (den2[:, None] + 1e-16) + bias2)

    aggp = _sc_agg(h, s2, d2)
    agg = (aggp[0] + aggp[1])[:N]
    return _final(h, agg, batch, Wrel, brel, Wroot, Wlin, blin)
